# Initial kernel scaffold; baseline (speedup 1.0000x reference)
import jax
import jax.numpy as jnp
from jax.experimental import pallas as pl


def kernel(x, edge_index, edge_attr, Wk, bk, Wq, bq, Wv, bv, Wa, ba, ln_g, ln_b):
    raise NotImplementedError("write your pallas kernel here")


# speedup vs baseline: 4.4449x; 4.4449x over previous
import jax
import jax.numpy as jnp
from jax.experimental import pallas as pl

N = 10000
T = 8
H = 4


def _noop_pallas(x):
    def body(x_ref, o_ref):
        o_ref[...] = x_ref[...]
    return pl.pallas_call(body, out_shape=jax.ShapeDtypeStruct(x.shape, x.dtype))(x)


def kernel(x, edge_index, edge_attr, Wk, bk, Wq, bq, Wv, bv, Wa, ba, ln_g, ln_b):
    OUT = Wk.shape[1]
    DK = OUT // H
    E = edge_index.shape[1]
    row = edge_index[0]
    col = edge_index[1]
    src_t = edge_attr[:, 3]
    trg_t = edge_attr[:, 4]

    inv = 1.0 / jnp.sqrt(jnp.float32(DK))
    Kp = (jnp.einsum('ni,toi->tno', x, Wk) + bk[:, None, :]) * inv
    Qp = jnp.einsum('ni,toi->tno', x, Wq) + bq[:, None, :]
    Vp = jnp.einsum('ni,toi->tno', x, Wv) + bv[:, None, :]

    KpF = Kp.reshape(T * N, OUT)
    QpF = Qp.reshape(T * N, OUT)
    VpF = Vp.reshape(T * N, OUT)
    i1 = src_t * N + row
    i2 = trg_t * N + col

    k1 = KpF[i1].reshape(-1, H, DK)
    q2 = QpF[i2].reshape(-1, H, DK)
    k2 = KpF[i2].reshape(-1, H, DK)
    q1 = QpF[i1].reshape(-1, H, DK)
    v1 = VpF[i1].reshape(-1, H, DK)
    v2 = VpF[i2].reshape(-1, H, DK)

    att = jnp.einsum('ehd,egd->eh', k1, q2)
    att_inv = jnp.einsum('ehd,egd->eh', k2, q1)
    f = jnp.exp(att)
    b = jnp.exp(att_inv)
    S_f = jax.ops.segment_sum(f, col, num_segments=N)
    S_b = jax.ops.segment_sum(b, row, num_segments=N)
    a_f = f / S_f[col]
    a_b = b / S_b[row]
    msg_f = (v1 * a_f[..., None]).reshape(-1, OUT)
    msg_b = (v2 * a_b[..., None]).reshape(-1, OUT)
    aggr = (jax.ops.segment_sum(msg_f, col, num_segments=N)
            + jax.ops.segment_sum(msg_b, row, num_segments=N))

    e_ids = jnp.arange(E, dtype=jnp.int32)
    pk_row = e_ids * 8 + src_t.astype(jnp.int32)
    pk_col = (e_ids + E) * 8 + trg_t.astype(jnp.int32)
    packed = jnp.full((N,), -1, dtype=jnp.int32)
    packed = packed.at[row].max(pk_row)
    packed = packed.at[col].max(pk_col)
    node_type = jnp.where(packed >= 0, packed & 7, 0).astype(src_t.dtype)

    g = jax.nn.gelu(aggr)
    Wa2 = Wa[:, :, :OUT] + Wa[:, :, OUT:]
    trans = jnp.einsum('no,tio->tni', g, Wa2) + ba[:, None, :]
    IN = x.shape[1]
    onehot = (node_type[None, :] == jnp.arange(T)[:, None]).astype(x.dtype)
    trans_sel = jnp.einsum('tn,tni->ni', onehot, trans)
    gam = jnp.einsum('tn,ti->ni', onehot, ln_g)
    bet = jnp.einsum('tn,ti->ni', onehot, ln_b)
    h = trans_sel + x
    mu = jnp.mean(h, axis=-1, keepdims=True)
    var = jnp.mean((h - mu) ** 2, axis=-1, keepdims=True)
    ln = (h - mu) / jnp.sqrt(var + 1e-5) * gam + bet
    res = ln
    return _noop_pallas(res)


# speedup vs baseline: 5.4888x; 1.2349x over previous
import functools
import math

import jax
import jax.numpy as jnp
from jax import lax
from jax.experimental import pallas as pl
from jax.experimental.pallas import tpu as pltpu
from jax.experimental.pallas import tpu_sc as plsc

N = 10000
T = 8
H = 4
IN = 128
OUT = 128
DK = OUT // H
E = 320000

NC = 2
NS = 16
NW = NC * NS
SUB = 80
EW = E // NW
NSUB = EW // SUB
ROWS = E // SUB


_BN = 1000


def _tables_body(x_ref, wk_ref, bk_ref, wq_ref, bq_ref, wv_ref, bv_ref,
                 kq_ref, v_ref):
    x = x_ref[...]
    t = pl.program_id(0)
    inv = 1.0 / math.sqrt(DK)
    bk = bk_ref[pl.ds(t, 1), :]
    bq = bq_ref[pl.ds(t, 1), :]
    bv = bv_ref[pl.ds(t, 1), :]
    k = (lax.dot_general(x, wk_ref[0], (((1,), (1,)), ((), ()))) + bk) * inv
    q = lax.dot_general(x, wq_ref[0], (((1,), (1,)), ((), ()))) + bq
    v = lax.dot_general(x, wv_ref[0], (((1,), (1,)), ((), ()))) + bv
    kq_ref[...] = jnp.concatenate([k, q], axis=1)
    v_ref[...] = v


def _make_tables(x, Wk, bk, Wq, bq, Wv, bv):
    nb = N // _BN
    return pl.pallas_call(
        _tables_body,
        grid=(T, nb),
        in_specs=[
            pl.BlockSpec((_BN, IN), lambda t, b: (b, 0)),
            pl.BlockSpec((1, OUT, IN), lambda t, b: (t, 0, 0)),
            pl.BlockSpec((T, OUT), lambda t, b: (0, 0)),
            pl.BlockSpec((1, OUT, IN), lambda t, b: (t, 0, 0)),
            pl.BlockSpec((T, OUT), lambda t, b: (0, 0)),
            pl.BlockSpec((1, OUT, IN), lambda t, b: (t, 0, 0)),
            pl.BlockSpec((T, OUT), lambda t, b: (0, 0)),
        ],
        out_specs=[
            pl.BlockSpec((_BN, 2 * OUT), lambda t, b: (t * (N // _BN) + b, 0)),
            pl.BlockSpec((_BN, OUT), lambda t, b: (t * (N // _BN) + b, 0)),
        ],
        out_shape=[
            jax.ShapeDtypeStruct((T * N, 2 * OUT), jnp.float32),
            jax.ShapeDtypeStruct((T * N, OUT), jnp.float32),
        ],
    )(x, Wk, bk, Wq, bq, Wv, bv)


SUB = 128
ROWS = E // SUB


def _chunk_range(w, per, extra):
    r0 = w * per + jnp.minimum(w, extra)
    trips = per + jnp.where(w < extra, 1, 0)
    return r0, trips


def _edge_att_body(kq_hbm, i1_hbm, i2_hbm, col_hbm, row_hbm, zeros_hbm,
                   fb_hbm, s_hbm,
                   i1_v, i2_v, col_v, row_v, kq1_b, kq2_b,
                   f_stage, b_stage, sfs, sbs, s_sp, sem1, sem2):
    c = lax.axis_index("c")
    s = lax.axis_index("s")
    wid = s * NC + c
    r0, trips = _chunk_range(wid, ROWS // NW, ROWS % NW)

    @pl.when(s == 0)
    def _():
        pltpu.sync_copy(zeros_hbm, s_sp)

    pltpu.sync_copy(zeros_hbm.at[pl.ds(0, SUB)], sfs)
    pltpu.sync_copy(zeros_hbm.at[pl.ds(0, SUB)], sbs)
    plsc.subcore_barrier()

    iota = lax.iota(jnp.int32, 16)

    def sub_body(sub, _):
        r = r0 + sub
        e0 = r * SUB
        pltpu.sync_copy(i1_hbm.at[pl.ds(e0, SUB)], i1_v)
        pltpu.sync_copy(i2_hbm.at[pl.ds(e0, SUB)], i2_v)
        pltpu.sync_copy(col_hbm.at[pl.ds(e0, SUB)], col_v)
        pltpu.sync_copy(row_hbm.at[pl.ds(e0, SUB)], row_v)
        cp1 = pltpu.async_copy(kq_hbm.at[i1_v], kq1_b, sem1)
        cp2 = pltpu.async_copy(kq_hbm.at[i2_v], kq2_b, sem2)
        cp1.wait()
        cp2.wait()

        def batch_body(j, _):
            eidx = iota + j * 16
            facc = [jnp.zeros((16,), jnp.float32) for _ in range(H)]
            bacc = [jnp.zeros((16,), jnp.float32) for _ in range(H)]
            for d in range(DK):
                qg1 = [plsc.load_gather(
                    kq1_b, [eidx, jnp.full((16,), OUT + g * DK + d, jnp.int32)])
                    for g in range(H)]
                qs1 = (qg1[0] + qg1[1]) + (qg1[2] + qg1[3])
                qg2 = [plsc.load_gather(
                    kq2_b, [eidx, jnp.full((16,), OUT + g * DK + d, jnp.int32)])
                    for g in range(H)]
                qs2 = (qg2[0] + qg2[1]) + (qg2[2] + qg2[3])
                for h in range(H):
                    ch = jnp.full((16,), h * DK + d, jnp.int32)
                    k1 = plsc.load_gather(kq1_b, [eidx, ch])
                    facc[h] = facc[h] + k1 * qs2
                    k2 = plsc.load_gather(kq2_b, [eidx, ch])
                    bacc[h] = bacc[h] + k2 * qs1
            for h in range(H):
                fh = jnp.exp(facc[h])
                bh = jnp.exp(bacc[h])
                f_stage[h, pl.ds(j * 16, 16)] = fh
                b_stage[h, pl.ds(j * 16, 16)] = bh
                ch = jnp.full((16,), h, jnp.int32)
                ch4 = jnp.full((16,), h + 4, jnp.int32)
                plsc.store_scatter(sfs, [eidx, ch], fh)
                plsc.store_scatter(sbs, [eidx, ch4], bh)
            return 0

        lax.fori_loop(0, SUB // 16, batch_body, 0)
        pltpu.sync_copy(f_stage, fb_hbm.at[0, :, pl.ds(e0, SUB)])
        pltpu.sync_copy(b_stage, fb_hbm.at[1, :, pl.ds(e0, SUB)])
        pltpu.sync_copy(sfs, s_sp.at[col_v], add=True)
        pltpu.sync_copy(sbs, s_sp.at[row_v], add=True)
        return 0

    lax.fori_loop(0, trips, sub_body, 0)
    plsc.subcore_barrier()

    @pl.when(s == 0)
    def _():
        pltpu.sync_copy(s_sp, s_hbm.at[c])


def _edge_att(KQ, i1, i2, col, row, zeros8):
    mesh = plsc.VectorSubcoreMesh(core_axis_name="c", subcore_axis_name="s")
    return pl.kernel(
        _edge_att_body,
        out_type=[
            jax.ShapeDtypeStruct((2, H, E), jnp.float32),
            jax.ShapeDtypeStruct((2, N, 8), jnp.float32),
        ],
        mesh=mesh,
        compiler_params=pltpu.CompilerParams(use_tc_tiling_on_sc=False, needs_layout_passes=False),
        scratch_types=[
            pltpu.VMEM((SUB,), jnp.int32),
            pltpu.VMEM((SUB,), jnp.int32),
            pltpu.VMEM((SUB,), jnp.int32),
            pltpu.VMEM((SUB,), jnp.int32),
            pltpu.VMEM((SUB, 2 * OUT), jnp.float32),
            pltpu.VMEM((SUB, 2 * OUT), jnp.float32),
            pltpu.VMEM((H, SUB), jnp.float32),
            pltpu.VMEM((H, SUB), jnp.float32),
            pltpu.VMEM((SUB, 8), jnp.float32),
            pltpu.VMEM((SUB, 8), jnp.float32),
            pltpu.VMEM_SHARED((N, 8), jnp.float32),
            pltpu.SemaphoreType.DMA,
            pltpu.SemaphoreType.DMA,
        ],
    )(KQ, i1, i2, col, row, zeros8)


def _msg_body(v_hbm, i1_hbm, i2_hbm, col_hbm, row_hbm, fb_hbm, zeros_hbm,
              u_hbm, idx_v, fb_v, v_b, msg, u_sp, sem1):
    c = lax.axis_index("c")
    s = lax.axis_index("s")
    r0, trips = _chunk_range(s, ROWS // NS, ROWS % NS)

    @pl.when(s == 0)
    def _():
        pltpu.sync_copy(zeros_hbm, u_sp)

    plsc.subcore_barrier()
    iota = lax.iota(jnp.int32, 16)

    def direction(d_ix, ix_hbm, seg_hbm):
        def sub_body(sub, _):
            r = r0 + sub
            e0 = r * SUB
            pltpu.sync_copy(ix_hbm.at[pl.ds(e0, SUB)], idx_v)
            pltpu.sync_copy(fb_hbm.at[d_ix, :, pl.ds(e0, SUB)], fb_v)
            pltpu.async_copy(v_hbm.at[idx_v], v_b, sem1).wait()

            def batch_body(j, _):
                eidx = iota + j * 16
                a = [fb_v[h, pl.ds(j * 16, 16)] for h in range(H)]
                for d in range(OUT):
                    cd = jnp.full((16,), d, jnp.int32)
                    vd = plsc.load_gather(v_b, [eidx, cd])
                    plsc.store_scatter(msg, [eidx, cd], vd * a[d // DK])
                return 0

            lax.fori_loop(0, SUB // 16, batch_body, 0)
            pltpu.sync_copy(seg_hbm.at[pl.ds(e0, SUB)], idx_v)
            pltpu.sync_copy(msg, u_sp.at[idx_v], add=True)
            return 0

        lax.fori_loop(0, trips, sub_body, 0)

    @pl.when(c == 0)
    def _():
        direction(0, i1_hbm, col_hbm)

    @pl.when(c == 1)
    def _():
        direction(1, i2_hbm, row_hbm)

    plsc.subcore_barrier()

    @pl.when(s == 0)
    def _():
        pltpu.sync_copy(u_sp, u_hbm.at[c])


def _msg_aggr(V, i1, i2, col, row, fb, zeros):
    mesh = plsc.VectorSubcoreMesh(core_axis_name="c", subcore_axis_name="s")
    return pl.kernel(
        _msg_body,
        out_type=jax.ShapeDtypeStruct((2, N, OUT), jnp.float32),
        mesh=mesh,
        compiler_params=pltpu.CompilerParams(use_tc_tiling_on_sc=False, needs_layout_passes=False),
        scratch_types=[
            pltpu.VMEM((SUB,), jnp.int32),
            pltpu.VMEM((H, SUB), jnp.float32),
            pltpu.VMEM((SUB, OUT), jnp.float32),
            pltpu.VMEM((SUB, OUT), jnp.float32),
            pltpu.VMEM_SHARED((N, OUT), jnp.float32),
            pltpu.SemaphoreType.DMA,
        ],
    )(V, i1, i2, col, row, fb, zeros)


def _gather16(x, idx):
    dn = lax.GatherDimensionNumbers(
        offset_dims=(), collapsed_slice_dims=(0,), start_index_map=(0,))
    return lax.gather(x, idx[:, None], dn, (1,),
                      mode=lax.GatherScatterMode.PROMISE_IN_BOUNDS)


def _node_type_body(row_hbm, col_hbm, st_hbm, tt_hbm, pk_hbm,
                    row_v, col_v, st_v, tt_v, local):
    c = lax.axis_index("c")
    s = lax.axis_index("s")
    wid = s * NC + c
    r0, trips = _chunk_range(wid, ROWS // NW, ROWS % NW)

    iota = lax.iota(jnp.int32, 16)
    neg1 = jnp.full((16,), -1, jnp.int32)

    def init_body(i, _):
        local[pl.ds(i * 16, 16)] = neg1
        return 0

    lax.fori_loop(0, N // 16, init_body, 0)

    perms = [(iota + k) % 16 for k in range(1, 16)]

    def upd(idx, val):
        for p in perms:
            oi = _gather16(idx, p)
            ov = _gather16(val, p)
            val = jnp.where(oi == idx, jnp.maximum(val, ov), val)
        cur = plsc.load_gather(local, [idx])
        plsc.store_scatter(local, [idx], jnp.maximum(cur, val))

    def sub_body(sub, _):
        r = r0 + sub
        e0 = r * SUB
        pltpu.sync_copy(row_hbm.at[pl.ds(e0, SUB)], row_v)
        pltpu.sync_copy(col_hbm.at[pl.ds(e0, SUB)], col_v)
        pltpu.sync_copy(st_hbm.at[pl.ds(e0, SUB)], st_v)
        pltpu.sync_copy(tt_hbm.at[pl.ds(e0, SUB)], tt_v)

        def batch_body(j, _):
            eid = e0 + j * 16 + iota
            ridx = row_v[pl.ds(j * 16, 16)]
            sval = st_v[pl.ds(j * 16, 16)]
            upd(ridx, eid * 8 + sval)
            cidx = col_v[pl.ds(j * 16, 16)]
            tval = tt_v[pl.ds(j * 16, 16)]
            upd(cidx, (eid + E) * 8 + tval)
            return 0

        lax.fori_loop(0, SUB // 16, batch_body, 0)
        return 0

    lax.fori_loop(0, trips, sub_body, 0)
    pltpu.sync_copy(local, pk_hbm.at[pl.ds(wid * N, N)])


def _node_type(row, col, st, tt):
    mesh = plsc.VectorSubcoreMesh(core_axis_name="c", subcore_axis_name="s")
    return pl.kernel(
        _node_type_body,
        out_type=jax.ShapeDtypeStruct((NW * N,), jnp.int32),
        mesh=mesh,
        compiler_params=pltpu.CompilerParams(use_tc_tiling_on_sc=False, needs_layout_passes=False),
        scratch_types=[
            pltpu.VMEM((SUB,), jnp.int32),
            pltpu.VMEM((SUB,), jnp.int32),
            pltpu.VMEM((SUB,), jnp.int32),
            pltpu.VMEM((SUB,), jnp.int32),
            pltpu.VMEM((N,), jnp.int32),
        ],
    )(row, col, st, tt)


_BF = 1000


def _final_body(u_ref, s_ref, packed_ref, x_ref, wa2_ref, ba_ref, lng_ref,
                lnb_ref, out_ref):
    S = s_ref[0] + s_ref[1]
    u0 = u_ref[0]
    u1 = u_ref[1]
    chunks = []
    for h in range(H):
        den_f = S[:, h:h + 1] + 1e-16
        den_b = S[:, h + 4:h + 5] + 1e-16
        chunks.append(u0[:, h * DK:(h + 1) * DK] / den_f
                      + u1[:, h * DK:(h + 1) * DK] / den_b)
    a = jnp.concatenate(chunks, axis=1)
    packed = jnp.max(packed_ref[0], axis=1, keepdims=True)
    nt = jnp.where(packed >= 0, packed & 7, 0)
    g = jax.nn.gelu(a)
    x = x_ref[...]
    acc = jnp.zeros_like(x)
    gam = jnp.zeros_like(x)
    bet = jnp.zeros_like(x)
    for t in range(T):
        yt = lax.dot_general(g, wa2_ref[t], (((1,), (1,)), ((), ()))) + ba_ref[t][None, :]
        sel = nt == t
        acc = jnp.where(sel, yt, acc)
        gam = jnp.where(sel, lng_ref[t][None, :], gam)
        bet = jnp.where(sel, lnb_ref[t][None, :], bet)
    h = acc + x
    mu = jnp.mean(h, axis=-1, keepdims=True)
    var = jnp.mean((h - mu) ** 2, axis=-1, keepdims=True)
    out_ref[...] = (h - mu) / jnp.sqrt(var + 1e-5) * gam + bet


def _final_stage(U, S_parts, packed_parts, x, Wa2, ba, ln_g, ln_b):
    nb = N // _BF
    P2 = packed_parts.shape[2]
    return pl.pallas_call(
        _final_body,
        grid=(nb,),
        in_specs=[
            pl.BlockSpec((2, _BF, IN), lambda b: (0, b, 0)),
            pl.BlockSpec((2, _BF, 8), lambda b: (0, b, 0)),
            pl.BlockSpec((1, _BF, P2), lambda b: (b, 0, 0)),
            pl.BlockSpec((_BF, IN), lambda b: (b, 0)),
            pl.BlockSpec((T, IN, OUT), lambda b: (0, 0, 0)),
            pl.BlockSpec((T, IN), lambda b: (0, 0)),
            pl.BlockSpec((T, IN), lambda b: (0, 0)),
            pl.BlockSpec((T, IN), lambda b: (0, 0)),
        ],
        out_specs=pl.BlockSpec((_BF, IN), lambda b: (b, 0)),
        out_shape=jax.ShapeDtypeStruct((N, IN), jnp.float32),
    )(U, S_parts, packed_parts, x, Wa2, ba, ln_g, ln_b)


def kernel(x, edge_index, edge_attr, Wk, bk, Wq, bq, Wv, bv, Wa, ba, ln_g, ln_b):
    row = edge_index[0].astype(jnp.int32)
    col = edge_index[1].astype(jnp.int32)
    src_t = edge_attr[:, 3].astype(jnp.int32)
    trg_t = edge_attr[:, 4].astype(jnp.int32)

    KQ, V = _make_tables(x, Wk, bk, Wq, bq, Wv, bv)

    i1 = src_t * N + row
    i2 = trg_t * N + col
    zeros = jnp.zeros((N, OUT), jnp.float32)
    zeros8 = jnp.zeros((N, 8), jnp.float32)

    fb, S_parts = _edge_att(KQ, i1, i2, col, row, zeros8)
    U = _msg_aggr(V, i1, i2, col, row, fb, zeros)
    pk = _node_type(row, col, src_t, trg_t)
    packed_parts = pk.reshape(NW, N // _BF, _BF).transpose(1, 2, 0)

    Wa2 = Wa[:, :, :OUT] + Wa[:, :, OUT:]
    return _final_stage(U, S_parts, packed_parts, x, Wa2, ba, ln_g, ln_b)


# speedup vs baseline: 14.5830x; 2.6569x over previous
import functools
import math

import jax
import jax.numpy as jnp
from jax import lax
from jax.experimental import pallas as pl
from jax.experimental.pallas import tpu as pltpu
from jax.experimental.pallas import tpu_sc as plsc

N = 10000
T = 8
H = 4
IN = 128
OUT = 128
DK = OUT // H
E = 320000

NC = 2
NS = 16
NW = NC * NS
SUB = 80
EW = E // NW
NSUB = EW // SUB
ROWS = E // SUB


_BN = 1000


def _tables_body(x_ref, wk_ref, bk_ref, wq_ref, bq_ref, wv_ref, bv_ref,
                 kq_ref, v_ref):
    x = x_ref[...]
    t = pl.program_id(0)
    inv = 1.0 / math.sqrt(DK)
    bk = bk_ref[pl.ds(t, 1), :]
    bq = bq_ref[pl.ds(t, 1), :]
    bv = bv_ref[pl.ds(t, 1), :]
    k = (lax.dot_general(x, wk_ref[0], (((1,), (1,)), ((), ()))) + bk) * inv
    q = lax.dot_general(x, wq_ref[0], (((1,), (1,)), ((), ()))) + bq
    v = lax.dot_general(x, wv_ref[0], (((1,), (1,)), ((), ()))) + bv
    kq_ref[...] = jnp.concatenate([k, q], axis=1)
    v_ref[...] = v


def _make_tables(x, Wk, bk, Wq, bq, Wv, bv):
    nb = N // _BN
    return pl.pallas_call(
        _tables_body,
        grid=(T, nb),
        in_specs=[
            pl.BlockSpec((_BN, IN), lambda t, b: (b, 0)),
            pl.BlockSpec((1, OUT, IN), lambda t, b: (t, 0, 0)),
            pl.BlockSpec((T, OUT), lambda t, b: (0, 0)),
            pl.BlockSpec((1, OUT, IN), lambda t, b: (t, 0, 0)),
            pl.BlockSpec((T, OUT), lambda t, b: (0, 0)),
            pl.BlockSpec((1, OUT, IN), lambda t, b: (t, 0, 0)),
            pl.BlockSpec((T, OUT), lambda t, b: (0, 0)),
        ],
        out_specs=[
            pl.BlockSpec((_BN, 2 * OUT), lambda t, b: (t * (N // _BN) + b, 0)),
            pl.BlockSpec((_BN, OUT), lambda t, b: (t * (N // _BN) + b, 0)),
        ],
        out_shape=[
            jax.ShapeDtypeStruct((T * N, 2 * OUT), jnp.float32),
            jax.ShapeDtypeStruct((T * N, OUT), jnp.float32),
        ],
    )(x, Wk, bk, Wq, bq, Wv, bv)


SUB = 128
ROWS = E // SUB


def _chunk_range(w, per, extra):
    r0 = w * per + jnp.minimum(w, extra)
    trips = per + jnp.where(w < extra, 1, 0)
    return r0, trips


def _edge_att_body(kq_hbm, i1_hbm, i2_hbm, col_hbm, row_hbm, zeros_hbm,
                   fb_hbm, s_hbm,
                   i1_v, i2_v, col_v, row_v, kq1_b, kq2_b,
                   f_stage, b_stage, sfs, sbs, s_sp, sem1, sem2):
    c = lax.axis_index("c")
    s = lax.axis_index("s")
    wid = s * NC + c
    r0, trips = _chunk_range(wid, ROWS // NW, ROWS % NW)

    @pl.when(s == 0)
    def _():
        pltpu.sync_copy(zeros_hbm, s_sp)

    pltpu.sync_copy(zeros_hbm.at[pl.ds(0, SUB)], sfs)
    pltpu.sync_copy(zeros_hbm.at[pl.ds(0, SUB)], sbs)
    plsc.subcore_barrier()

    iota = lax.iota(jnp.int32, 16)

    def sub_body(sub, _):
        r = r0 + sub
        e0 = r * SUB
        pltpu.sync_copy(i1_hbm.at[pl.ds(e0, SUB)], i1_v)
        pltpu.sync_copy(i2_hbm.at[pl.ds(e0, SUB)], i2_v)
        pltpu.sync_copy(col_hbm.at[pl.ds(e0, SUB)], col_v)
        pltpu.sync_copy(row_hbm.at[pl.ds(e0, SUB)], row_v)
        cp1 = pltpu.async_copy(kq_hbm.at[i1_v], kq1_b, sem1)
        cp2 = pltpu.async_copy(kq_hbm.at[i2_v], kq2_b, sem2)
        cp1.wait()
        cp2.wait()

        def batch_body(j, _):
            eidx = iota + j * 16
            facc = [jnp.zeros((16,), jnp.float32) for _ in range(H)]
            bacc = [jnp.zeros((16,), jnp.float32) for _ in range(H)]
            for d in range(DK):
                rot = (iota + d) & (DK - 1)
                qg1 = [plsc.load_gather(kq1_b, [eidx, rot + (OUT + g * DK)])
                       for g in range(H)]
                qs1 = (qg1[0] + qg1[1]) + (qg1[2] + qg1[3])
                qg2 = [plsc.load_gather(kq2_b, [eidx, rot + (OUT + g * DK)])
                       for g in range(H)]
                qs2 = (qg2[0] + qg2[1]) + (qg2[2] + qg2[3])
                for h in range(H):
                    ch = rot + h * DK
                    k1 = plsc.load_gather(kq1_b, [eidx, ch])
                    facc[h] = facc[h] + k1 * qs2
                    k2 = plsc.load_gather(kq2_b, [eidx, ch])
                    bacc[h] = bacc[h] + k2 * qs1
            for h in range(H):
                fh = jnp.exp(facc[h])
                bh = jnp.exp(bacc[h])
                f_stage[h, pl.ds(j * 16, 16)] = fh
                b_stage[h, pl.ds(j * 16, 16)] = bh
                ch = jnp.full((16,), h, jnp.int32)
                ch4 = jnp.full((16,), h + 4, jnp.int32)
                plsc.store_scatter(sfs, [eidx, ch], fh)
                plsc.store_scatter(sbs, [eidx, ch4], bh)
            return 0

        lax.fori_loop(0, SUB // 16, batch_body, 0)
        pltpu.sync_copy(f_stage, fb_hbm.at[0, :, pl.ds(e0, SUB)])
        pltpu.sync_copy(b_stage, fb_hbm.at[1, :, pl.ds(e0, SUB)])
        pltpu.sync_copy(sfs, s_sp.at[col_v], add=True)
        pltpu.sync_copy(sbs, s_sp.at[row_v], add=True)
        return 0

    lax.fori_loop(0, trips, sub_body, 0)
    plsc.subcore_barrier()

    @pl.when(s == 0)
    def _():
        pltpu.sync_copy(s_sp, s_hbm.at[c])


def _edge_att(KQ, i1, i2, col, row, zeros8):
    mesh = plsc.VectorSubcoreMesh(core_axis_name="c", subcore_axis_name="s")
    return pl.kernel(
        _edge_att_body,
        out_type=[
            jax.ShapeDtypeStruct((2, H, E), jnp.float32),
            jax.ShapeDtypeStruct((2, N, 8), jnp.float32),
        ],
        mesh=mesh,
        compiler_params=pltpu.CompilerParams(use_tc_tiling_on_sc=False, needs_layout_passes=False),
        scratch_types=[
            pltpu.VMEM((SUB,), jnp.int32),
            pltpu.VMEM((SUB,), jnp.int32),
            pltpu.VMEM((SUB,), jnp.int32),
            pltpu.VMEM((SUB,), jnp.int32),
            pltpu.VMEM((SUB, 2 * OUT), jnp.float32),
            pltpu.VMEM((SUB, 2 * OUT), jnp.float32),
            pltpu.VMEM((H, SUB), jnp.float32),
            pltpu.VMEM((H, SUB), jnp.float32),
            pltpu.VMEM((SUB, 8), jnp.float32),
            pltpu.VMEM((SUB, 8), jnp.float32),
            pltpu.VMEM_SHARED((N, 8), jnp.float32),
            pltpu.SemaphoreType.DMA,
            pltpu.SemaphoreType.DMA,
        ],
    )(KQ, i1, i2, col, row, zeros8)


SUB2 = 128
ROWS2 = E // SUB2


def _msg_body(v_hbm, i1_hbm, i2_hbm, col_hbm, row_hbm, fb_hbm, zeros_hbm,
              u_hbm, idx_v, fb_v, v_b, msg, u_sp, sem1):
    c = lax.axis_index("c")
    s = lax.axis_index("s")
    r0, trips = _chunk_range(s, ROWS2 // NS, ROWS2 % NS)

    @pl.when(s == 0)
    def _():
        pltpu.sync_copy(zeros_hbm, u_sp)

    plsc.subcore_barrier()
    iota = lax.iota(jnp.int32, 16)

    def direction(d_ix, ix_hbm, seg_hbm):
        def sub_body(sub, _):
            r = r0 + sub
            e0 = r * SUB2
            pltpu.sync_copy(ix_hbm.at[pl.ds(e0, SUB2)], idx_v)
            pltpu.sync_copy(fb_hbm.at[d_ix, :, pl.ds(e0, SUB2)], fb_v)
            pltpu.async_copy(v_hbm.at[idx_v], v_b, sem1).wait()

            def batch_body(j, _):
                eidx = iota + j * 16
                a = [fb_v[h, pl.ds(j * 16, 16)] for h in range(H)]
                for d in range(DK):
                    rot = (iota + d) & (DK - 1)
                    for h in range(H):
                        cd = rot + h * DK
                        vd = plsc.load_gather(v_b, [eidx, cd])
                        plsc.store_scatter(msg, [eidx, cd], vd * a[h])
                return 0

            lax.fori_loop(0, SUB2 // 16, batch_body, 0)
            pltpu.sync_copy(seg_hbm.at[pl.ds(e0, SUB2)], idx_v)
            pltpu.sync_copy(msg, u_sp.at[idx_v], add=True)
            return 0

        lax.fori_loop(0, trips, sub_body, 0)

    @pl.when(c == 0)
    def _():
        direction(0, i1_hbm, col_hbm)

    @pl.when(c == 1)
    def _():
        direction(1, i2_hbm, row_hbm)

    plsc.subcore_barrier()

    @pl.when(s == 0)
    def _():
        pltpu.sync_copy(u_sp, u_hbm.at[c])


def _msg_aggr(V, i1, i2, col, row, fb, zeros):
    mesh = plsc.VectorSubcoreMesh(core_axis_name="c", subcore_axis_name="s")
    return pl.kernel(
        _msg_body,
        out_type=jax.ShapeDtypeStruct((2, N, OUT), jnp.float32),
        mesh=mesh,
        compiler_params=pltpu.CompilerParams(use_tc_tiling_on_sc=False, needs_layout_passes=False),
        scratch_types=[
            pltpu.VMEM((SUB2,), jnp.int32),
            pltpu.VMEM((H, SUB2), jnp.float32),
            pltpu.VMEM((SUB2, OUT), jnp.float32),
            pltpu.VMEM((SUB2, OUT), jnp.float32),
            pltpu.VMEM_SHARED((N, OUT), jnp.float32),
            pltpu.SemaphoreType.DMA,
        ],
    )(V, i1, i2, col, row, fb, zeros)


def _gather16(x, idx):
    dn = lax.GatherDimensionNumbers(
        offset_dims=(), collapsed_slice_dims=(0,), start_index_map=(0,))
    return lax.gather(x, idx[:, None], dn, (1,),
                      mode=lax.GatherScatterMode.PROMISE_IN_BOUNDS)


def _node_type_body(row_hbm, col_hbm, st_hbm, tt_hbm, pk_hbm,
                    row_v, col_v, st_v, tt_v, local):
    c = lax.axis_index("c")
    s = lax.axis_index("s")
    wid = s * NC + c
    r0, trips = _chunk_range(wid, ROWS // NW, ROWS % NW)

    iota = lax.iota(jnp.int32, 16)
    neg1 = jnp.full((16,), -1, jnp.int32)

    def init_body(i, _):
        local[pl.ds(i * 16, 16)] = neg1
        return 0

    lax.fori_loop(0, N // 16, init_body, 0)

    perms = [(iota + k) % 16 for k in range(1, 16)]

    def upd(idx, val):
        for p in perms:
            oi = _gather16(idx, p)
            ov = _gather16(val, p)
            val = jnp.where(oi == idx, jnp.maximum(val, ov), val)
        cur = plsc.load_gather(local, [idx])
        plsc.store_scatter(local, [idx], jnp.maximum(cur, val))

    def sub_body(sub, _):
        r = r0 + sub
        e0 = r * SUB
        pltpu.sync_copy(row_hbm.at[pl.ds(e0, SUB)], row_v)
        pltpu.sync_copy(col_hbm.at[pl.ds(e0, SUB)], col_v)
        pltpu.sync_copy(st_hbm.at[pl.ds(e0, SUB)], st_v)
        pltpu.sync_copy(tt_hbm.at[pl.ds(e0, SUB)], tt_v)

        def batch_body(j, _):
            eid = e0 + j * 16 + iota
            ridx = row_v[pl.ds(j * 16, 16)]
            sval = st_v[pl.ds(j * 16, 16)]
            upd(ridx, eid * 8 + sval)
            cidx = col_v[pl.ds(j * 16, 16)]
            tval = tt_v[pl.ds(j * 16, 16)]
            upd(cidx, (eid + E) * 8 + tval)
            return 0

        lax.fori_loop(0, SUB // 16, batch_body, 0)
        return 0

    lax.fori_loop(0, trips, sub_body, 0)
    pltpu.sync_copy(local, pk_hbm.at[pl.ds(wid * N, N)])


def _node_type(row, col, st, tt):
    mesh = plsc.VectorSubcoreMesh(core_axis_name="c", subcore_axis_name="s")
    return pl.kernel(
        _node_type_body,
        out_type=jax.ShapeDtypeStruct((NW * N,), jnp.int32),
        mesh=mesh,
        compiler_params=pltpu.CompilerParams(use_tc_tiling_on_sc=False, needs_layout_passes=False),
        scratch_types=[
            pltpu.VMEM((SUB,), jnp.int32),
            pltpu.VMEM((SUB,), jnp.int32),
            pltpu.VMEM((SUB,), jnp.int32),
            pltpu.VMEM((SUB,), jnp.int32),
            pltpu.VMEM((N,), jnp.int32),
        ],
    )(row, col, st, tt)


_BF = 1000


def _final_body(u_ref, s_ref, packed_ref, x_ref, wa2_ref, ba_ref, lng_ref,
                lnb_ref, out_ref):
    S = s_ref[0] + s_ref[1]
    u0 = u_ref[0]
    u1 = u_ref[1]
    chunks = []
    for h in range(H):
        den_f = S[:, h:h + 1] + 1e-16
        den_b = S[:, h + 4:h + 5] + 1e-16
        chunks.append(u0[:, h * DK:(h + 1) * DK] / den_f
                      + u1[:, h * DK:(h + 1) * DK] / den_b)
    a = jnp.concatenate(chunks, axis=1)
    packed = jnp.max(packed_ref[0], axis=1, keepdims=True)
    nt = jnp.where(packed >= 0, packed & 7, 0)
    g = jax.nn.gelu(a)
    x = x_ref[...]
    acc = jnp.zeros_like(x)
    gam = jnp.zeros_like(x)
    bet = jnp.zeros_like(x)
    for t in range(T):
        yt = lax.dot_general(g, wa2_ref[t], (((1,), (1,)), ((), ()))) + ba_ref[t][None, :]
        sel = nt == t
        acc = jnp.where(sel, yt, acc)
        gam = jnp.where(sel, lng_ref[t][None, :], gam)
        bet = jnp.where(sel, lnb_ref[t][None, :], bet)
    h = acc + x
    mu = jnp.mean(h, axis=-1, keepdims=True)
    var = jnp.mean((h - mu) ** 2, axis=-1, keepdims=True)
    out_ref[...] = (h - mu) / jnp.sqrt(var + 1e-5) * gam + bet


def _final_stage(U, S_parts, packed_parts, x, Wa2, ba, ln_g, ln_b):
    nb = N // _BF
    P2 = packed_parts.shape[2]
    return pl.pallas_call(
        _final_body,
        grid=(nb,),
        in_specs=[
            pl.BlockSpec((2, _BF, IN), lambda b: (0, b, 0)),
            pl.BlockSpec((2, _BF, 8), lambda b: (0, b, 0)),
            pl.BlockSpec((1, _BF, P2), lambda b: (b, 0, 0)),
            pl.BlockSpec((_BF, IN), lambda b: (b, 0)),
            pl.BlockSpec((T, IN, OUT), lambda b: (0, 0, 0)),
            pl.BlockSpec((T, IN), lambda b: (0, 0)),
            pl.BlockSpec((T, IN), lambda b: (0, 0)),
            pl.BlockSpec((T, IN), lambda b: (0, 0)),
        ],
        out_specs=pl.BlockSpec((_BF, IN), lambda b: (b, 0)),
        out_shape=jax.ShapeDtypeStruct((N, IN), jnp.float32),
    )(U, S_parts, packed_parts, x, Wa2, ba, ln_g, ln_b)


def kernel(x, edge_index, edge_attr, Wk, bk, Wq, bq, Wv, bv, Wa, ba, ln_g, ln_b):
    row = edge_index[0].astype(jnp.int32)
    col = edge_index[1].astype(jnp.int32)
    src_t = edge_attr[:, 3].astype(jnp.int32)
    trg_t = edge_attr[:, 4].astype(jnp.int32)

    KQ, V = _make_tables(x, Wk, bk, Wq, bq, Wv, bv)

    i1 = src_t * N + row
    i2 = trg_t * N + col
    zeros = jnp.zeros((N, OUT), jnp.float32)
    zeros8 = jnp.zeros((N, 8), jnp.float32)

    fb, S_parts = _edge_att(KQ, i1, i2, col, row, zeros8)
    U = _msg_aggr(V, i1, i2, col, row, fb, zeros)
    pk = _node_type(row, col, src_t, trg_t)
    packed_parts = pk.reshape(NW, N // _BF, _BF).transpose(1, 2, 0)

    Wa2 = Wa[:, :, :OUT] + Wa[:, :, OUT:]
    return _final_stage(U, S_parts, packed_parts, x, Wa2, ba, ln_g, ln_b)


# speedup vs baseline: 15.1146x; 1.0365x over previous
import functools
import math

import jax
import jax.numpy as jnp
from jax import lax
from jax.experimental import pallas as pl
from jax.experimental.pallas import tpu as pltpu
from jax.experimental.pallas import tpu_sc as plsc

N = 10000
T = 8
H = 4
IN = 128
OUT = 128
DK = OUT // H
E = 320000

NC = 2
NS = 16
NW = NC * NS
SUB = 80
EW = E // NW
NSUB = EW // SUB
ROWS = E // SUB


_BN = 1000


def _tables_body(x_ref, wk_ref, bk_ref, wq_ref, bq_ref, wv_ref, bv_ref,
                 kq_ref, v_ref):
    x = x_ref[...]
    t = pl.program_id(0)
    inv = 1.0 / math.sqrt(DK)
    bk = bk_ref[pl.ds(t, 1), :]
    bq = bq_ref[pl.ds(t, 1), :]
    bv = bv_ref[pl.ds(t, 1), :]
    k = (lax.dot_general(x, wk_ref[0], (((1,), (1,)), ((), ()))) + bk) * inv
    q = lax.dot_general(x, wq_ref[0], (((1,), (1,)), ((), ()))) + bq
    v = lax.dot_general(x, wv_ref[0], (((1,), (1,)), ((), ()))) + bv
    kq_ref[...] = jnp.concatenate([k, q], axis=1)
    v_ref[...] = v


def _make_tables(x, Wk, bk, Wq, bq, Wv, bv):
    nb = N // _BN
    return pl.pallas_call(
        _tables_body,
        grid=(T, nb),
        in_specs=[
            pl.BlockSpec((_BN, IN), lambda t, b: (b, 0)),
            pl.BlockSpec((1, OUT, IN), lambda t, b: (t, 0, 0)),
            pl.BlockSpec((T, OUT), lambda t, b: (0, 0)),
            pl.BlockSpec((1, OUT, IN), lambda t, b: (t, 0, 0)),
            pl.BlockSpec((T, OUT), lambda t, b: (0, 0)),
            pl.BlockSpec((1, OUT, IN), lambda t, b: (t, 0, 0)),
            pl.BlockSpec((T, OUT), lambda t, b: (0, 0)),
        ],
        out_specs=[
            pl.BlockSpec((_BN, 2 * OUT), lambda t, b: (t * (N // _BN) + b, 0)),
            pl.BlockSpec((_BN, OUT), lambda t, b: (t * (N // _BN) + b, 0)),
        ],
        out_shape=[
            jax.ShapeDtypeStruct((T * N, 2 * OUT), jnp.float32),
            jax.ShapeDtypeStruct((T * N, OUT), jnp.float32),
        ],
    )(x, Wk, bk, Wq, bq, Wv, bv)


SUB = 128
ROWS = E // SUB


def _chunk_range(w, per, extra):
    r0 = w * per + jnp.minimum(w, extra)
    trips = per + jnp.where(w < extra, 1, 0)
    return r0, trips


def _edge_att_body(kq_hbm, i1_hbm, i2_hbm, col_hbm, row_hbm, zeros_hbm,
                   fb_hbm, s_hbm,
                   i1_v, i2_v, col_v, row_v, kq1_b, kq2_b,
                   f_stage, b_stage, sfs, sbs, s_sp, sem1, sem2):
    c = lax.axis_index("c")
    s = lax.axis_index("s")
    wid = s * NC + c
    r0, trips = _chunk_range(wid, ROWS // NW, ROWS % NW)

    @pl.when(s == 0)
    def _():
        pltpu.sync_copy(zeros_hbm, s_sp)

    pltpu.sync_copy(zeros_hbm.at[pl.ds(0, SUB)], sfs)
    pltpu.sync_copy(zeros_hbm.at[pl.ds(0, SUB)], sbs)
    plsc.subcore_barrier()

    iota = lax.iota(jnp.int32, 16)

    def sub_body(sub, _):
        r = r0 + sub
        e0 = r * SUB
        pltpu.sync_copy(i1_hbm.at[pl.ds(e0, SUB)], i1_v)
        pltpu.sync_copy(i2_hbm.at[pl.ds(e0, SUB)], i2_v)
        pltpu.sync_copy(col_hbm.at[pl.ds(e0, SUB)], col_v)
        pltpu.sync_copy(row_hbm.at[pl.ds(e0, SUB)], row_v)
        cp1 = pltpu.async_copy(kq_hbm.at[i1_v], kq1_b, sem1)
        cp2 = pltpu.async_copy(kq_hbm.at[i2_v], kq2_b, sem2)
        cp1.wait()
        cp2.wait()

        def batch_body(j, _):
            eidx = iota + j * 16
            facc = [jnp.zeros((16,), jnp.float32) for _ in range(H)]
            bacc = [jnp.zeros((16,), jnp.float32) for _ in range(H)]
            for d in range(DK):
                rot = (iota + d) & (DK - 1)
                qg1 = [plsc.load_gather(kq1_b, [eidx, rot + (OUT + g * DK)])
                       for g in range(H)]
                qs1 = (qg1[0] + qg1[1]) + (qg1[2] + qg1[3])
                qg2 = [plsc.load_gather(kq2_b, [eidx, rot + (OUT + g * DK)])
                       for g in range(H)]
                qs2 = (qg2[0] + qg2[1]) + (qg2[2] + qg2[3])
                for h in range(H):
                    ch = rot + h * DK
                    k1 = plsc.load_gather(kq1_b, [eidx, ch])
                    facc[h] = facc[h] + k1 * qs2
                    k2 = plsc.load_gather(kq2_b, [eidx, ch])
                    bacc[h] = bacc[h] + k2 * qs1
            for h in range(H):
                fh = jnp.exp(facc[h])
                bh = jnp.exp(bacc[h])
                f_stage[h, pl.ds(j * 16, 16)] = fh
                b_stage[h, pl.ds(j * 16, 16)] = bh
                ch = jnp.full((16,), h, jnp.int32)
                ch4 = jnp.full((16,), h + 4, jnp.int32)
                plsc.store_scatter(sfs, [eidx, ch], fh)
                plsc.store_scatter(sbs, [eidx, ch4], bh)
            return 0

        lax.fori_loop(0, SUB // 16, batch_body, 0)
        pltpu.sync_copy(f_stage, fb_hbm.at[0, :, pl.ds(e0, SUB)])
        pltpu.sync_copy(b_stage, fb_hbm.at[1, :, pl.ds(e0, SUB)])
        pltpu.sync_copy(sfs, s_sp.at[col_v], add=True)
        pltpu.sync_copy(sbs, s_sp.at[row_v], add=True)
        return 0

    lax.fori_loop(0, trips, sub_body, 0)
    plsc.subcore_barrier()

    @pl.when(s == 0)
    def _():
        pltpu.sync_copy(s_sp, s_hbm.at[c])


def _edge_att(KQ, i1, i2, col, row, zeros8):
    mesh = plsc.VectorSubcoreMesh(core_axis_name="c", subcore_axis_name="s")
    return pl.kernel(
        _edge_att_body,
        out_type=[
            jax.ShapeDtypeStruct((2, H, E), jnp.float32),
            jax.ShapeDtypeStruct((2, N, 8), jnp.float32),
        ],
        mesh=mesh,
        compiler_params=pltpu.CompilerParams(use_tc_tiling_on_sc=False, needs_layout_passes=False),
        scratch_types=[
            pltpu.VMEM((SUB,), jnp.int32),
            pltpu.VMEM((SUB,), jnp.int32),
            pltpu.VMEM((SUB,), jnp.int32),
            pltpu.VMEM((SUB,), jnp.int32),
            pltpu.VMEM((SUB, 2 * OUT), jnp.float32),
            pltpu.VMEM((SUB, 2 * OUT), jnp.float32),
            pltpu.VMEM((H, SUB), jnp.float32),
            pltpu.VMEM((H, SUB), jnp.float32),
            pltpu.VMEM((SUB, 8), jnp.float32),
            pltpu.VMEM((SUB, 8), jnp.float32),
            pltpu.VMEM_SHARED((N, 8), jnp.float32),
            pltpu.SemaphoreType.DMA,
            pltpu.SemaphoreType.DMA,
        ],
    )(KQ, i1, i2, col, row, zeros8)


SUB2 = 128
ROWS2 = E // SUB2


def _msg_body(v_hbm, i1_hbm, i2_hbm, col_hbm, row_hbm, fb_hbm, zeros_hbm,
              u_hbm, idx_v, seg0, fb_v, v_b, msg0, u_sp,
              sem1, sema0):
    c = lax.axis_index("c")
    s = lax.axis_index("s")
    r0, trips = _chunk_range(s, ROWS2 // NS, ROWS2 % NS)

    @pl.when(s == 0)
    def _():
        pltpu.sync_copy(zeros_hbm, u_sp)

    plsc.subcore_barrier()
    iota = lax.iota(jnp.int32, 16)

    def direction(d_ix, ix_hbm, seg_hbm):
        def sub_body(sub, _):
            r = r0 + sub
            e0 = r * SUB2
            pltpu.sync_copy(ix_hbm.at[pl.ds(e0, SUB2)], idx_v)
            pltpu.sync_copy(fb_hbm.at[d_ix, :, pl.ds(e0, SUB2)], fb_v)
            cpv = pltpu.async_copy(v_hbm.at[idx_v], v_b, sem1)
            @pl.when(sub >= 1)
            def _():
                pltpu.make_async_copy(msg0, u_sp.at[seg0], sema0).wait()

            cpv.wait()

            def batch_body(j, _):
                eidx = iota + j * 16
                a = [fb_v[h, pl.ds(j * 16, 16)] for h in range(H)]
                for d in range(DK):
                    rot = (iota + d) & (DK - 1)
                    for h in range(H):
                        cd = rot + h * DK
                        vd = plsc.load_gather(v_b, [eidx, cd])
                        plsc.store_scatter(msg0, [eidx, cd], vd * a[h])
                return 0

            lax.fori_loop(0, SUB2 // 16, batch_body, 0)
            pltpu.sync_copy(seg_hbm.at[pl.ds(e0, SUB2)], seg0)
            pltpu.async_copy(msg0, u_sp.at[seg0], sema0, add=True)
            return 0

        lax.fori_loop(0, trips, sub_body, 0)
        pltpu.make_async_copy(msg0, u_sp.at[seg0], sema0).wait()

    @pl.when(c == 0)
    def _():
        direction(0, i1_hbm, col_hbm)

    @pl.when(c == 1)
    def _():
        direction(1, i2_hbm, row_hbm)

    plsc.subcore_barrier()

    @pl.when(s == 0)
    def _():
        pltpu.sync_copy(u_sp, u_hbm.at[c])


def _msg_aggr(V, i1, i2, col, row, fb, zeros):
    mesh = plsc.VectorSubcoreMesh(core_axis_name="c", subcore_axis_name="s")
    return pl.kernel(
        _msg_body,
        out_type=jax.ShapeDtypeStruct((2, N, OUT), jnp.float32),
        mesh=mesh,
        compiler_params=pltpu.CompilerParams(use_tc_tiling_on_sc=False, needs_layout_passes=False),
        scratch_types=[
            pltpu.VMEM((SUB2,), jnp.int32),
            pltpu.VMEM((SUB2,), jnp.int32),
            pltpu.VMEM((H, SUB2), jnp.float32),
            pltpu.VMEM((SUB2, OUT), jnp.float32),
            pltpu.VMEM((SUB2, OUT), jnp.float32),
            pltpu.VMEM_SHARED((N, OUT), jnp.float32),
            pltpu.SemaphoreType.DMA,
            pltpu.SemaphoreType.DMA,
        ],
    )(V, i1, i2, col, row, fb, zeros)


def _gather16(x, idx):
    dn = lax.GatherDimensionNumbers(
        offset_dims=(), collapsed_slice_dims=(0,), start_index_map=(0,))
    return lax.gather(x, idx[:, None], dn, (1,),
                      mode=lax.GatherScatterMode.PROMISE_IN_BOUNDS)


def _node_type_body(row_hbm, col_hbm, st_hbm, tt_hbm, pk_hbm,
                    row_v, col_v, st_v, tt_v, local):
    c = lax.axis_index("c")
    s = lax.axis_index("s")
    wid = s * NC + c
    r0, trips = _chunk_range(wid, ROWS // NW, ROWS % NW)

    iota = lax.iota(jnp.int32, 16)
    neg1 = jnp.full((16,), -1, jnp.int32)

    def init_body(i, _):
        local[pl.ds(i * 16, 16)] = neg1
        return 0

    lax.fori_loop(0, N // 16, init_body, 0)

    perms = [(iota + k) % 16 for k in range(1, 16)]

    def upd(idx, val):
        for p in perms:
            oi = _gather16(idx, p)
            ov = _gather16(val, p)
            val = jnp.where(oi == idx, jnp.maximum(val, ov), val)
        cur = plsc.load_gather(local, [idx])
        plsc.store_scatter(local, [idx], jnp.maximum(cur, val))

    def sub_body(sub, _):
        r = r0 + sub
        e0 = r * SUB
        pltpu.sync_copy(row_hbm.at[pl.ds(e0, SUB)], row_v)
        pltpu.sync_copy(col_hbm.at[pl.ds(e0, SUB)], col_v)
        pltpu.sync_copy(st_hbm.at[pl.ds(e0, SUB)], st_v)
        pltpu.sync_copy(tt_hbm.at[pl.ds(e0, SUB)], tt_v)

        def batch_body(j, _):
            eid = e0 + j * 16 + iota
            ridx = row_v[pl.ds(j * 16, 16)]
            sval = st_v[pl.ds(j * 16, 16)]
            upd(ridx, eid * 8 + sval)
            cidx = col_v[pl.ds(j * 16, 16)]
            tval = tt_v[pl.ds(j * 16, 16)]
            upd(cidx, (eid + E) * 8 + tval)
            return 0

        lax.fori_loop(0, SUB // 16, batch_body, 0)
        return 0

    lax.fori_loop(0, trips, sub_body, 0)
    pltpu.sync_copy(local, pk_hbm.at[pl.ds(wid * N, N)])


def _node_type(row, col, st, tt):
    mesh = plsc.VectorSubcoreMesh(core_axis_name="c", subcore_axis_name="s")
    return pl.kernel(
        _node_type_body,
        out_type=jax.ShapeDtypeStruct((NW * N,), jnp.int32),
        mesh=mesh,
        compiler_params=pltpu.CompilerParams(use_tc_tiling_on_sc=False, needs_layout_passes=False),
        scratch_types=[
            pltpu.VMEM((SUB,), jnp.int32),
            pltpu.VMEM((SUB,), jnp.int32),
            pltpu.VMEM((SUB,), jnp.int32),
            pltpu.VMEM((SUB,), jnp.int32),
            pltpu.VMEM((N,), jnp.int32),
        ],
    )(row, col, st, tt)


_BF = 1000


def _final_body(u_ref, s_ref, packed_ref, x_ref, wa2_ref, ba_ref, lng_ref,
                lnb_ref, out_ref):
    S = s_ref[0] + s_ref[1]
    u0 = u_ref[0]
    u1 = u_ref[1]
    chunks = []
    for h in range(H):
        den_f = S[:, h:h + 1] + 1e-16
        den_b = S[:, h + 4:h + 5] + 1e-16
        chunks.append(u0[:, h * DK:(h + 1) * DK] / den_f
                      + u1[:, h * DK:(h + 1) * DK] / den_b)
    a = jnp.concatenate(chunks, axis=1)
    packed = jnp.max(packed_ref[0], axis=1, keepdims=True)
    nt = jnp.where(packed >= 0, packed & 7, 0)
    g = jax.nn.gelu(a)
    x = x_ref[...]
    acc = jnp.zeros_like(x)
    gam = jnp.zeros_like(x)
    bet = jnp.zeros_like(x)
    for t in range(T):
        yt = lax.dot_general(g, wa2_ref[t], (((1,), (1,)), ((), ()))) + ba_ref[t][None, :]
        sel = nt == t
        acc = jnp.where(sel, yt, acc)
        gam = jnp.where(sel, lng_ref[t][None, :], gam)
        bet = jnp.where(sel, lnb_ref[t][None, :], bet)
    h = acc + x
    mu = jnp.mean(h, axis=-1, keepdims=True)
    var = jnp.mean((h - mu) ** 2, axis=-1, keepdims=True)
    out_ref[...] = (h - mu) / jnp.sqrt(var + 1e-5) * gam + bet


def _final_stage(U, S_parts, packed_parts, x, Wa2, ba, ln_g, ln_b):
    nb = N // _BF
    P2 = packed_parts.shape[2]
    return pl.pallas_call(
        _final_body,
        grid=(nb,),
        in_specs=[
            pl.BlockSpec((2, _BF, IN), lambda b: (0, b, 0)),
            pl.BlockSpec((2, _BF, 8), lambda b: (0, b, 0)),
            pl.BlockSpec((1, _BF, P2), lambda b: (b, 0, 0)),
            pl.BlockSpec((_BF, IN), lambda b: (b, 0)),
            pl.BlockSpec((T, IN, OUT), lambda b: (0, 0, 0)),
            pl.BlockSpec((T, IN), lambda b: (0, 0)),
            pl.BlockSpec((T, IN), lambda b: (0, 0)),
            pl.BlockSpec((T, IN), lambda b: (0, 0)),
        ],
        out_specs=pl.BlockSpec((_BF, IN), lambda b: (b, 0)),
        out_shape=jax.ShapeDtypeStruct((N, IN), jnp.float32),
    )(U, S_parts, packed_parts, x, Wa2, ba, ln_g, ln_b)


def kernel(x, edge_index, edge_attr, Wk, bk, Wq, bq, Wv, bv, Wa, ba, ln_g, ln_b):
    row = edge_index[0].astype(jnp.int32)
    col = edge_index[1].astype(jnp.int32)
    src_t = edge_attr[:, 3].astype(jnp.int32)
    trg_t = edge_attr[:, 4].astype(jnp.int32)

    KQ, V = _make_tables(x, Wk, bk, Wq, bq, Wv, bv)

    i1 = src_t * N + row
    i2 = trg_t * N + col
    zeros = jnp.zeros((N, OUT), jnp.float32)
    zeros8 = jnp.zeros((N, 8), jnp.float32)

    fb, S_parts = _edge_att(KQ, i1, i2, col, row, zeros8)
    U = _msg_aggr(V, i1, i2, col, row, fb, zeros)
    pk = _node_type(row, col, src_t, trg_t)
    packed_parts = pk.reshape(NW, N // _BF, _BF).transpose(1, 2, 0)

    Wa2 = Wa[:, :, :OUT] + Wa[:, :, OUT:]
    return _final_stage(U, S_parts, packed_parts, x, Wa2, ba, ln_g, ln_b)


# speedup vs baseline: 15.6204x; 1.0335x over previous
import functools
import math

import jax
import jax.numpy as jnp
from jax import lax
from jax.experimental import pallas as pl
from jax.experimental.pallas import tpu as pltpu
from jax.experimental.pallas import tpu_sc as plsc

N = 10000
T = 8
H = 4
IN = 128
OUT = 128
DK = OUT // H
E = 320000

NC = 2
NS = 16
NW = NC * NS
SUB = 80
EW = E // NW
NSUB = EW // SUB
ROWS = E // SUB


_BN = 1000


def _tables_body(x_ref, wk_ref, bk_ref, wq_ref, bq_ref, wv_ref, bv_ref,
                 kq_ref, v_ref):
    x = x_ref[...]
    t = pl.program_id(0)
    inv = 1.0 / math.sqrt(DK)
    bk = bk_ref[pl.ds(t, 1), :]
    bq = bq_ref[pl.ds(t, 1), :]
    bv = bv_ref[pl.ds(t, 1), :]
    k = (lax.dot_general(x, wk_ref[0], (((1,), (1,)), ((), ()))) + bk) * inv
    q = lax.dot_general(x, wq_ref[0], (((1,), (1,)), ((), ()))) + bq
    v = lax.dot_general(x, wv_ref[0], (((1,), (1,)), ((), ()))) + bv
    kq_ref[...] = jnp.concatenate([k, q], axis=1)
    v_ref[...] = v


def _make_tables(x, Wk, bk, Wq, bq, Wv, bv):
    nb = N // _BN
    return pl.pallas_call(
        _tables_body,
        grid=(T, nb),
        in_specs=[
            pl.BlockSpec((_BN, IN), lambda t, b: (b, 0)),
            pl.BlockSpec((1, OUT, IN), lambda t, b: (t, 0, 0)),
            pl.BlockSpec((T, OUT), lambda t, b: (0, 0)),
            pl.BlockSpec((1, OUT, IN), lambda t, b: (t, 0, 0)),
            pl.BlockSpec((T, OUT), lambda t, b: (0, 0)),
            pl.BlockSpec((1, OUT, IN), lambda t, b: (t, 0, 0)),
            pl.BlockSpec((T, OUT), lambda t, b: (0, 0)),
        ],
        out_specs=[
            pl.BlockSpec((_BN, 2 * OUT), lambda t, b: (t * (N // _BN) + b, 0)),
            pl.BlockSpec((_BN, OUT), lambda t, b: (t * (N // _BN) + b, 0)),
        ],
        out_shape=[
            jax.ShapeDtypeStruct((T * N, 2 * OUT), jnp.float32),
            jax.ShapeDtypeStruct((T * N, OUT), jnp.float32),
        ],
    )(x, Wk, bk, Wq, bq, Wv, bv)


SUB = 128
ROWS = E // SUB


def _chunk_range(w, per, extra):
    r0 = w * per + jnp.minimum(w, extra)
    trips = per + jnp.where(w < extra, 1, 0)
    return r0, trips


def _edge_att_body(kq_hbm, i1_hbm, i2_hbm, col_hbm, row_hbm, zeros_hbm,
                   fb_hbm, s_hbm,
                   i1_v, i2_v, col_v, row_v, kq1_b, kq2_b,
                   f_stage, b_stage, sfs, sbs, s_sp, sem1, sem2):
    c = lax.axis_index("c")
    s = lax.axis_index("s")
    wid = s * NC + c
    r0, trips = _chunk_range(wid, ROWS // NW, ROWS % NW)

    @pl.when(s == 0)
    def _():
        pltpu.sync_copy(zeros_hbm, s_sp)

    pltpu.sync_copy(zeros_hbm.at[pl.ds(0, SUB)], sfs)
    pltpu.sync_copy(zeros_hbm.at[pl.ds(0, SUB)], sbs)
    plsc.subcore_barrier()

    iota = lax.iota(jnp.int32, 16)

    def sub_body(sub, _):
        r = r0 + sub
        e0 = r * SUB
        pltpu.sync_copy(i1_hbm.at[pl.ds(e0, SUB)], i1_v)
        pltpu.sync_copy(i2_hbm.at[pl.ds(e0, SUB)], i2_v)
        pltpu.sync_copy(col_hbm.at[pl.ds(e0, SUB)], col_v)
        pltpu.sync_copy(row_hbm.at[pl.ds(e0, SUB)], row_v)
        cp1 = pltpu.async_copy(kq_hbm.at[i1_v], kq1_b, sem1)
        cp2 = pltpu.async_copy(kq_hbm.at[i2_v], kq2_b, sem2)
        cp1.wait()
        cp2.wait()

        def batch_body(j, _):
            eidx = iota + j * 16
            facc = [jnp.zeros((16,), jnp.float32) for _ in range(H)]
            bacc = [jnp.zeros((16,), jnp.float32) for _ in range(H)]
            for d in range(DK):
                rot = (iota + d) & (DK - 1)
                qg1 = [plsc.load_gather(kq1_b, [eidx, rot + (OUT + g * DK)])
                       for g in range(H)]
                qs1 = (qg1[0] + qg1[1]) + (qg1[2] + qg1[3])
                qg2 = [plsc.load_gather(kq2_b, [eidx, rot + (OUT + g * DK)])
                       for g in range(H)]
                qs2 = (qg2[0] + qg2[1]) + (qg2[2] + qg2[3])
                for h in range(H):
                    ch = rot + h * DK
                    k1 = plsc.load_gather(kq1_b, [eidx, ch])
                    facc[h] = facc[h] + k1 * qs2
                    k2 = plsc.load_gather(kq2_b, [eidx, ch])
                    bacc[h] = bacc[h] + k2 * qs1
            for h in range(H):
                fh = jnp.exp(facc[h])
                bh = jnp.exp(bacc[h])
                f_stage[h, pl.ds(j * 16, 16)] = fh
                b_stage[h, pl.ds(j * 16, 16)] = bh
                ch = jnp.full((16,), h, jnp.int32)
                ch4 = jnp.full((16,), h + 4, jnp.int32)
                plsc.store_scatter(sfs, [eidx, ch], fh)
                plsc.store_scatter(sbs, [eidx, ch4], bh)
            return 0

        lax.fori_loop(0, SUB // 16, batch_body, 0)
        pltpu.sync_copy(f_stage, fb_hbm.at[0, :, pl.ds(e0, SUB)])
        pltpu.sync_copy(b_stage, fb_hbm.at[1, :, pl.ds(e0, SUB)])
        pltpu.sync_copy(sfs, s_sp.at[col_v], add=True)
        pltpu.sync_copy(sbs, s_sp.at[row_v], add=True)
        return 0

    lax.fori_loop(0, trips, sub_body, 0)
    plsc.subcore_barrier()

    @pl.when(s == 0)
    def _():
        pltpu.sync_copy(s_sp, s_hbm.at[c])


def _edge_att(KQ, i1, i2, col, row, zeros8):
    mesh = plsc.VectorSubcoreMesh(core_axis_name="c", subcore_axis_name="s")
    return pl.kernel(
        _edge_att_body,
        out_type=[
            jax.ShapeDtypeStruct((2, H, E), jnp.float32),
            jax.ShapeDtypeStruct((2, N, 8), jnp.float32),
        ],
        mesh=mesh,
        compiler_params=pltpu.CompilerParams(use_tc_tiling_on_sc=False, needs_layout_passes=False),
        scratch_types=[
            pltpu.VMEM((SUB,), jnp.int32),
            pltpu.VMEM((SUB,), jnp.int32),
            pltpu.VMEM((SUB,), jnp.int32),
            pltpu.VMEM((SUB,), jnp.int32),
            pltpu.VMEM((SUB, 2 * OUT), jnp.float32),
            pltpu.VMEM((SUB, 2 * OUT), jnp.float32),
            pltpu.VMEM((H, SUB), jnp.float32),
            pltpu.VMEM((H, SUB), jnp.float32),
            pltpu.VMEM((SUB, 8), jnp.float32),
            pltpu.VMEM((SUB, 8), jnp.float32),
            pltpu.VMEM_SHARED((N, 8), jnp.float32),
            pltpu.SemaphoreType.DMA,
            pltpu.SemaphoreType.DMA,
        ],
    )(KQ, i1, i2, col, row, zeros8)


SUB2 = 128
ROWS2 = E // SUB2


def _msg_body(v_hbm, i1_hbm, i2_hbm, col_hbm, row_hbm, fb_hbm, zeros_hbm,
              u_hbm, idx_v, seg, fb_v, v_b, u_sp,
              semg0, semg1, sema0, sema1):
    c = lax.axis_index("c")
    s = lax.axis_index("s")
    r0, trips = _chunk_range(s, ROWS2 // NS, ROWS2 % NS)

    @pl.when(s == 0)
    def _():
        pltpu.sync_copy(zeros_hbm, u_sp)

    plsc.subcore_barrier()
    iota = lax.iota(jnp.int32, 16)

    def direction(d_ix, ix_hbm, seg_hbm):
        def issue(sub, p):
            e0 = (r0 + sub) * SUB2
            pltpu.sync_copy(ix_hbm.at[pl.ds(e0, SUB2)], idx_v.at[p])
            if p == 0:
                pltpu.async_copy(v_hbm.at[idx_v.at[0]], v_b.at[0], semg0)
            else:
                pltpu.async_copy(v_hbm.at[idx_v.at[1]], v_b.at[1], semg1)

        issue(0, 0)

        def sub_body(sub, _):
            r = r0 + sub
            e0 = r * SUB2
            p = sub % 2

            @pl.when((sub + 1 < trips) & (p == 0))
            def _():
                @pl.when(sub >= 1)
                def _():
                    pltpu.make_async_copy(v_b.at[1], u_sp.at[seg.at[1]],
                                          sema1).wait()
                issue(sub + 1, 1)

            @pl.when((sub + 1 < trips) & (p == 1))
            def _():
                pltpu.make_async_copy(v_b.at[0], u_sp.at[seg.at[0]],
                                      sema0).wait()
                issue(sub + 1, 0)

            @pl.when(p == 0)
            def _():
                pltpu.make_async_copy(v_hbm.at[idx_v.at[0]], v_b.at[0],
                                      semg0).wait()

            @pl.when(p == 1)
            def _():
                pltpu.make_async_copy(v_hbm.at[idx_v.at[1]], v_b.at[1],
                                      semg1).wait()

            pltpu.sync_copy(fb_hbm.at[d_ix, :, pl.ds(e0, SUB2)], fb_v)
            pvec = jnp.full((16,), p, jnp.int32)

            def batch_body(j, _):
                eidx = iota + j * 16
                a = [fb_v[h, pl.ds(j * 16, 16)] for h in range(H)]
                for d in range(DK):
                    rot = (iota + d) & (DK - 1)
                    for h in range(H):
                        cd = rot + h * DK
                        vd = plsc.load_gather(v_b, [pvec, eidx, cd])
                        plsc.store_scatter(v_b, [pvec, eidx, cd], vd * a[h])
                return 0

            lax.fori_loop(0, SUB2 // 16, batch_body, 0)

            @pl.when(p == 0)
            def _():
                pltpu.sync_copy(seg_hbm.at[pl.ds(e0, SUB2)], seg.at[0])
                pltpu.async_copy(v_b.at[0], u_sp.at[seg.at[0]], sema0,
                                 add=True)

            @pl.when(p == 1)
            def _():
                pltpu.sync_copy(seg_hbm.at[pl.ds(e0, SUB2)], seg.at[1])
                pltpu.async_copy(v_b.at[1], u_sp.at[seg.at[1]], sema1,
                                 add=True)

            return 0

        lax.fori_loop(0, trips, sub_body, 0)
        pltpu.make_async_copy(v_b.at[0], u_sp.at[seg.at[0]], sema0).wait()
        pltpu.make_async_copy(v_b.at[1], u_sp.at[seg.at[1]], sema1).wait()

    @pl.when(c == 0)
    def _():
        direction(0, i1_hbm, col_hbm)

    @pl.when(c == 1)
    def _():
        direction(1, i2_hbm, row_hbm)

    plsc.subcore_barrier()

    @pl.when(s == 0)
    def _():
        pltpu.sync_copy(u_sp, u_hbm.at[c])


def _msg_aggr(V, i1, i2, col, row, fb, zeros):
    mesh = plsc.VectorSubcoreMesh(core_axis_name="c", subcore_axis_name="s")
    return pl.kernel(
        _msg_body,
        out_type=jax.ShapeDtypeStruct((2, N, OUT), jnp.float32),
        mesh=mesh,
        compiler_params=pltpu.CompilerParams(use_tc_tiling_on_sc=False, needs_layout_passes=False),
        scratch_types=[
            pltpu.VMEM((2, SUB2), jnp.int32),
            pltpu.VMEM((2, SUB2), jnp.int32),
            pltpu.VMEM((H, SUB2), jnp.float32),
            pltpu.VMEM((2, SUB2, OUT), jnp.float32),
            pltpu.VMEM_SHARED((N, OUT), jnp.float32),
            pltpu.SemaphoreType.DMA,
            pltpu.SemaphoreType.DMA,
            pltpu.SemaphoreType.DMA,
            pltpu.SemaphoreType.DMA,
        ],
    )(V, i1, i2, col, row, fb, zeros)


def _gather16(x, idx):
    dn = lax.GatherDimensionNumbers(
        offset_dims=(), collapsed_slice_dims=(0,), start_index_map=(0,))
    return lax.gather(x, idx[:, None], dn, (1,),
                      mode=lax.GatherScatterMode.PROMISE_IN_BOUNDS)


def _node_type_body(row_hbm, col_hbm, st_hbm, tt_hbm, pk_hbm,
                    row_v, col_v, st_v, tt_v, local):
    c = lax.axis_index("c")
    s = lax.axis_index("s")
    wid = s * NC + c
    r0, trips = _chunk_range(wid, ROWS // NW, ROWS % NW)

    iota = lax.iota(jnp.int32, 16)
    neg1 = jnp.full((16,), -1, jnp.int32)

    def init_body(i, _):
        local[pl.ds(i * 16, 16)] = neg1
        return 0

    lax.fori_loop(0, N // 16, init_body, 0)

    perms = [(iota + k) % 16 for k in range(1, 16)]

    def upd(idx, val):
        for p in perms:
            oi = _gather16(idx, p)
            ov = _gather16(val, p)
            val = jnp.where(oi == idx, jnp.maximum(val, ov), val)
        cur = plsc.load_gather(local, [idx])
        plsc.store_scatter(local, [idx], jnp.maximum(cur, val))

    def sub_body(sub, _):
        r = r0 + sub
        e0 = r * SUB
        pltpu.sync_copy(row_hbm.at[pl.ds(e0, SUB)], row_v)
        pltpu.sync_copy(col_hbm.at[pl.ds(e0, SUB)], col_v)
        pltpu.sync_copy(st_hbm.at[pl.ds(e0, SUB)], st_v)
        pltpu.sync_copy(tt_hbm.at[pl.ds(e0, SUB)], tt_v)

        def batch_body(j, _):
            eid = e0 + j * 16 + iota
            ridx = row_v[pl.ds(j * 16, 16)]
            sval = st_v[pl.ds(j * 16, 16)]
            upd(ridx, eid * 8 + sval)
            cidx = col_v[pl.ds(j * 16, 16)]
            tval = tt_v[pl.ds(j * 16, 16)]
            upd(cidx, (eid + E) * 8 + tval)
            return 0

        lax.fori_loop(0, SUB // 16, batch_body, 0)
        return 0

    lax.fori_loop(0, trips, sub_body, 0)
    pltpu.sync_copy(local, pk_hbm.at[pl.ds(wid * N, N)])


def _node_type(row, col, st, tt):
    mesh = plsc.VectorSubcoreMesh(core_axis_name="c", subcore_axis_name="s")
    return pl.kernel(
        _node_type_body,
        out_type=jax.ShapeDtypeStruct((NW * N,), jnp.int32),
        mesh=mesh,
        compiler_params=pltpu.CompilerParams(use_tc_tiling_on_sc=False, needs_layout_passes=False),
        scratch_types=[
            pltpu.VMEM((SUB,), jnp.int32),
            pltpu.VMEM((SUB,), jnp.int32),
            pltpu.VMEM((SUB,), jnp.int32),
            pltpu.VMEM((SUB,), jnp.int32),
            pltpu.VMEM((N,), jnp.int32),
        ],
    )(row, col, st, tt)


_BF = 1000


def _final_body(u_ref, s_ref, packed_ref, x_ref, wa2_ref, ba_ref, lng_ref,
                lnb_ref, out_ref):
    S = s_ref[0] + s_ref[1]
    u0 = u_ref[0]
    u1 = u_ref[1]
    chunks = []
    for h in range(H):
        den_f = S[:, h:h + 1] + 1e-16
        den_b = S[:, h + 4:h + 5] + 1e-16
        chunks.append(u0[:, h * DK:(h + 1) * DK] / den_f
                      + u1[:, h * DK:(h + 1) * DK] / den_b)
    a = jnp.concatenate(chunks, axis=1)
    packed = jnp.max(packed_ref[0], axis=1, keepdims=True)
    nt = jnp.where(packed >= 0, packed & 7, 0)
    g = jax.nn.gelu(a)
    x = x_ref[...]
    acc = jnp.zeros_like(x)
    gam = jnp.zeros_like(x)
    bet = jnp.zeros_like(x)
    for t in range(T):
        yt = lax.dot_general(g, wa2_ref[t], (((1,), (1,)), ((), ()))) + ba_ref[t][None, :]
        sel = nt == t
        acc = jnp.where(sel, yt, acc)
        gam = jnp.where(sel, lng_ref[t][None, :], gam)
        bet = jnp.where(sel, lnb_ref[t][None, :], bet)
    h = acc + x
    mu = jnp.mean(h, axis=-1, keepdims=True)
    var = jnp.mean((h - mu) ** 2, axis=-1, keepdims=True)
    out_ref[...] = (h - mu) / jnp.sqrt(var + 1e-5) * gam + bet


def _final_stage(U, S_parts, packed_parts, x, Wa2, ba, ln_g, ln_b):
    nb = N // _BF
    P2 = packed_parts.shape[2]
    return pl.pallas_call(
        _final_body,
        grid=(nb,),
        in_specs=[
            pl.BlockSpec((2, _BF, IN), lambda b: (0, b, 0)),
            pl.BlockSpec((2, _BF, 8), lambda b: (0, b, 0)),
            pl.BlockSpec((1, _BF, P2), lambda b: (b, 0, 0)),
            pl.BlockSpec((_BF, IN), lambda b: (b, 0)),
            pl.BlockSpec((T, IN, OUT), lambda b: (0, 0, 0)),
            pl.BlockSpec((T, IN), lambda b: (0, 0)),
            pl.BlockSpec((T, IN), lambda b: (0, 0)),
            pl.BlockSpec((T, IN), lambda b: (0, 0)),
        ],
        out_specs=pl.BlockSpec((_BF, IN), lambda b: (b, 0)),
        out_shape=jax.ShapeDtypeStruct((N, IN), jnp.float32),
    )(U, S_parts, packed_parts, x, Wa2, ba, ln_g, ln_b)


def kernel(x, edge_index, edge_attr, Wk, bk, Wq, bq, Wv, bv, Wa, ba, ln_g, ln_b):
    row = edge_index[0].astype(jnp.int32)
    col = edge_index[1].astype(jnp.int32)
    src_t = edge_attr[:, 3].astype(jnp.int32)
    trg_t = edge_attr[:, 4].astype(jnp.int32)

    KQ, V = _make_tables(x, Wk, bk, Wq, bq, Wv, bv)

    i1 = src_t * N + row
    i2 = trg_t * N + col
    zeros = jnp.zeros((N, OUT), jnp.float32)
    zeros8 = jnp.zeros((N, 8), jnp.float32)

    fb, S_parts = _edge_att(KQ, i1, i2, col, row, zeros8)
    U = _msg_aggr(V, i1, i2, col, row, fb, zeros)
    pk = _node_type(row, col, src_t, trg_t)
    packed_parts = pk.reshape(NW, N // _BF, _BF).transpose(1, 2, 0)

    Wa2 = Wa[:, :, :OUT] + Wa[:, :, OUT:]
    return _final_stage(U, S_parts, packed_parts, x, Wa2, ba, ln_g, ln_b)


# speedup vs baseline: 16.3039x; 1.0438x over previous
import functools
import math

import jax
import jax.numpy as jnp
from jax import lax
from jax.experimental import pallas as pl
from jax.experimental.pallas import tpu as pltpu
from jax.experimental.pallas import tpu_sc as plsc

N = 10000
T = 8
H = 4
IN = 128
OUT = 128
DK = OUT // H
E = 320000

NC = 2
NS = 16
NW = NC * NS
SUB = 80
EW = E // NW
NSUB = EW // SUB
ROWS = E // SUB


_BN = 1000


def _tables_body(x_ref, wk_ref, bk_ref, wq_ref, bq_ref, wv_ref, bv_ref,
                 kq_ref, v_ref):
    x = x_ref[...]
    t = pl.program_id(0)
    inv = 1.0 / math.sqrt(DK)
    bk = bk_ref[pl.ds(t, 1), :]
    bq = bq_ref[pl.ds(t, 1), :]
    bv = bv_ref[pl.ds(t, 1), :]
    k = (lax.dot_general(x, wk_ref[0], (((1,), (1,)), ((), ()))) + bk) * inv
    q = lax.dot_general(x, wq_ref[0], (((1,), (1,)), ((), ()))) + bq
    v = lax.dot_general(x, wv_ref[0], (((1,), (1,)), ((), ()))) + bv
    kq_ref[...] = jnp.concatenate([k, q], axis=1)
    v_ref[...] = v


def _make_tables(x, Wk, bk, Wq, bq, Wv, bv):
    nb = N // _BN
    return pl.pallas_call(
        _tables_body,
        grid=(T, nb),
        in_specs=[
            pl.BlockSpec((_BN, IN), lambda t, b: (b, 0)),
            pl.BlockSpec((1, OUT, IN), lambda t, b: (t, 0, 0)),
            pl.BlockSpec((T, OUT), lambda t, b: (0, 0)),
            pl.BlockSpec((1, OUT, IN), lambda t, b: (t, 0, 0)),
            pl.BlockSpec((T, OUT), lambda t, b: (0, 0)),
            pl.BlockSpec((1, OUT, IN), lambda t, b: (t, 0, 0)),
            pl.BlockSpec((T, OUT), lambda t, b: (0, 0)),
        ],
        out_specs=[
            pl.BlockSpec((_BN, 2 * OUT), lambda t, b: (t * (N // _BN) + b, 0)),
            pl.BlockSpec((_BN, OUT), lambda t, b: (t * (N // _BN) + b, 0)),
        ],
        out_shape=[
            jax.ShapeDtypeStruct((T * N, 2 * OUT), jnp.float32),
            jax.ShapeDtypeStruct((T * N, OUT), jnp.float32),
        ],
    )(x, Wk, bk, Wq, bq, Wv, bv)


SUB = 128
ROWS = E // SUB


def _chunk_range(w, per, extra):
    r0 = w * per + jnp.minimum(w, extra)
    trips = per + jnp.where(w < extra, 1, 0)
    return r0, trips


def _edge_att_body(kq_hbm, idx4_hbm, zeros_hbm,
                   fb_hbm, s_hbm,
                   idx4_v, kq1_b, kq2_b,
                   f_stage, b_stage, sfs, sbs, s_sp, sem1, sem2):
    c = lax.axis_index("c")
    s = lax.axis_index("s")
    wid = s * NC + c
    r0, trips = _chunk_range(wid, ROWS // NW, ROWS % NW)

    @pl.when(s == 0)
    def _():
        pltpu.sync_copy(zeros_hbm, s_sp)

    pltpu.sync_copy(zeros_hbm.at[pl.ds(0, SUB)], sfs)
    pltpu.sync_copy(zeros_hbm.at[pl.ds(0, SUB)], sbs)
    plsc.subcore_barrier()

    iota = lax.iota(jnp.int32, 16)

    def sub_body(sub, _):
        r = r0 + sub
        e0 = r * SUB
        pltpu.sync_copy(idx4_hbm.at[:, pl.ds(e0, SUB)], idx4_v)
        cp1 = pltpu.async_copy(kq_hbm.at[idx4_v.at[0]], kq1_b, sem1)
        cp2 = pltpu.async_copy(kq_hbm.at[idx4_v.at[1]], kq2_b, sem2)
        cp1.wait()
        cp2.wait()

        def batch_body(j, _):
            eidx = iota + j * 16
            facc = [jnp.zeros((16,), jnp.float32) for _ in range(H)]
            bacc = [jnp.zeros((16,), jnp.float32) for _ in range(H)]
            for d in range(DK):
                rot = (iota + d) & (DK - 1)
                qg1 = [plsc.load_gather(kq1_b, [eidx, rot + (OUT + g * DK)])
                       for g in range(H)]
                qs1 = (qg1[0] + qg1[1]) + (qg1[2] + qg1[3])
                qg2 = [plsc.load_gather(kq2_b, [eidx, rot + (OUT + g * DK)])
                       for g in range(H)]
                qs2 = (qg2[0] + qg2[1]) + (qg2[2] + qg2[3])
                for h in range(H):
                    ch = rot + h * DK
                    k1 = plsc.load_gather(kq1_b, [eidx, ch])
                    facc[h] = facc[h] + k1 * qs2
                    k2 = plsc.load_gather(kq2_b, [eidx, ch])
                    bacc[h] = bacc[h] + k2 * qs1
            for h in range(H):
                fh = jnp.exp(facc[h])
                bh = jnp.exp(bacc[h])
                f_stage[h, pl.ds(j * 16, 16)] = fh
                b_stage[h, pl.ds(j * 16, 16)] = bh
                ch = jnp.full((16,), h, jnp.int32)
                ch4 = jnp.full((16,), h + 4, jnp.int32)
                plsc.store_scatter(sfs, [eidx, ch], fh)
                plsc.store_scatter(sbs, [eidx, ch4], bh)
            return 0

        lax.fori_loop(0, SUB // 16, batch_body, 0)
        pltpu.sync_copy(f_stage, fb_hbm.at[0, :, pl.ds(e0, SUB)])
        pltpu.sync_copy(b_stage, fb_hbm.at[1, :, pl.ds(e0, SUB)])
        pltpu.sync_copy(sfs, s_sp.at[idx4_v.at[2]], add=True)
        pltpu.sync_copy(sbs, s_sp.at[idx4_v.at[3]], add=True)
        return 0

    lax.fori_loop(0, trips, sub_body, 0)
    plsc.subcore_barrier()

    @pl.when(s == 0)
    def _():
        pltpu.sync_copy(s_sp, s_hbm.at[c])


def _edge_att(KQ, idx4, zeros8):
    mesh = plsc.VectorSubcoreMesh(core_axis_name="c", subcore_axis_name="s")
    return pl.kernel(
        _edge_att_body,
        out_type=[
            jax.ShapeDtypeStruct((2, H, E), jnp.float32),
            jax.ShapeDtypeStruct((2, N, 8), jnp.float32),
        ],
        mesh=mesh,
        compiler_params=pltpu.CompilerParams(use_tc_tiling_on_sc=False, needs_layout_passes=False),
        scratch_types=[
            pltpu.VMEM((4, SUB), jnp.int32),
            pltpu.VMEM((SUB, 2 * OUT), jnp.float32),
            pltpu.VMEM((SUB, 2 * OUT), jnp.float32),
            pltpu.VMEM((H, SUB), jnp.float32),
            pltpu.VMEM((H, SUB), jnp.float32),
            pltpu.VMEM((SUB, 8), jnp.float32),
            pltpu.VMEM((SUB, 8), jnp.float32),
            pltpu.VMEM_SHARED((N, 8), jnp.float32),
            pltpu.SemaphoreType.DMA,
            pltpu.SemaphoreType.DMA,
        ],
    )(KQ, idx4, zeros8)


SUB2 = 128
ROWS2 = E // SUB2


def _msg_body(v_hbm, i1_hbm, i2_hbm, col_hbm, row_hbm, fb_hbm, zeros_hbm,
              u_hbm, idx_v, seg, fb_v, v_b, u_sp,
              semg0, semg1, sema0, sema1):
    c = lax.axis_index("c")
    s = lax.axis_index("s")
    r0, trips = _chunk_range(s, ROWS2 // NS, ROWS2 % NS)

    @pl.when(s == 0)
    def _():
        pltpu.sync_copy(zeros_hbm, u_sp)

    plsc.subcore_barrier()
    iota = lax.iota(jnp.int32, 16)

    def direction(d_ix, ix_hbm, seg_hbm):
        def issue(sub, p):
            e0 = (r0 + sub) * SUB2
            pltpu.sync_copy(ix_hbm.at[pl.ds(e0, SUB2)], idx_v.at[p])
            if p == 0:
                pltpu.async_copy(v_hbm.at[idx_v.at[0]], v_b.at[0], semg0)
            else:
                pltpu.async_copy(v_hbm.at[idx_v.at[1]], v_b.at[1], semg1)

        issue(0, 0)

        def sub_body(sub, _):
            r = r0 + sub
            e0 = r * SUB2
            p = sub % 2

            @pl.when((sub + 1 < trips) & (p == 0))
            def _():
                @pl.when(sub >= 1)
                def _():
                    pltpu.make_async_copy(v_b.at[1], u_sp.at[seg.at[1]],
                                          sema1).wait()
                issue(sub + 1, 1)

            @pl.when((sub + 1 < trips) & (p == 1))
            def _():
                pltpu.make_async_copy(v_b.at[0], u_sp.at[seg.at[0]],
                                      sema0).wait()
                issue(sub + 1, 0)

            @pl.when(p == 0)
            def _():
                pltpu.make_async_copy(v_hbm.at[idx_v.at[0]], v_b.at[0],
                                      semg0).wait()

            @pl.when(p == 1)
            def _():
                pltpu.make_async_copy(v_hbm.at[idx_v.at[1]], v_b.at[1],
                                      semg1).wait()

            pltpu.sync_copy(fb_hbm.at[d_ix, :, pl.ds(e0, SUB2)], fb_v)
            vbp = v_b.at[p]

            def batch_body(j, _):
                eidx = iota + j * 16
                a = [fb_v[h, pl.ds(j * 16, 16)] for h in range(H)]
                for d in range(DK):
                    rot = (iota + d) & (DK - 1)
                    for h in range(H):
                        cd = rot + h * DK
                        vd = plsc.load_gather(vbp, [eidx, cd])
                        plsc.store_scatter(vbp, [eidx, cd], vd * a[h])
                return 0

            lax.fori_loop(0, SUB2 // 16, batch_body, 0)

            @pl.when(p == 0)
            def _():
                pltpu.sync_copy(seg_hbm.at[pl.ds(e0, SUB2)], seg.at[0])
                pltpu.async_copy(v_b.at[0], u_sp.at[seg.at[0]], sema0,
                                 add=True)

            @pl.when(p == 1)
            def _():
                pltpu.sync_copy(seg_hbm.at[pl.ds(e0, SUB2)], seg.at[1])
                pltpu.async_copy(v_b.at[1], u_sp.at[seg.at[1]], sema1,
                                 add=True)

            return 0

        lax.fori_loop(0, trips, sub_body, 0)
        pltpu.make_async_copy(v_b.at[0], u_sp.at[seg.at[0]], sema0).wait()
        pltpu.make_async_copy(v_b.at[1], u_sp.at[seg.at[1]], sema1).wait()

    @pl.when(c == 0)
    def _():
        direction(0, i1_hbm, col_hbm)

    @pl.when(c == 1)
    def _():
        direction(1, i2_hbm, row_hbm)

    plsc.subcore_barrier()

    @pl.when(s == 0)
    def _():
        pltpu.sync_copy(u_sp, u_hbm.at[c])


def _msg_aggr(V, i1, i2, col, row, fb, zeros):
    mesh = plsc.VectorSubcoreMesh(core_axis_name="c", subcore_axis_name="s")
    return pl.kernel(
        _msg_body,
        out_type=jax.ShapeDtypeStruct((2, N, OUT), jnp.float32),
        mesh=mesh,
        compiler_params=pltpu.CompilerParams(use_tc_tiling_on_sc=False, needs_layout_passes=False),
        scratch_types=[
            pltpu.VMEM((2, SUB2), jnp.int32),
            pltpu.VMEM((2, SUB2), jnp.int32),
            pltpu.VMEM((H, SUB2), jnp.float32),
            pltpu.VMEM((2, SUB2, OUT), jnp.float32),
            pltpu.VMEM_SHARED((N, OUT), jnp.float32),
            pltpu.SemaphoreType.DMA,
            pltpu.SemaphoreType.DMA,
            pltpu.SemaphoreType.DMA,
            pltpu.SemaphoreType.DMA,
        ],
    )(V, i1, i2, col, row, fb, zeros)


def _gather16(x, idx):
    dn = lax.GatherDimensionNumbers(
        offset_dims=(), collapsed_slice_dims=(0,), start_index_map=(0,))
    return lax.gather(x, idx[:, None], dn, (1,),
                      mode=lax.GatherScatterMode.PROMISE_IN_BOUNDS)


def _node_type_body(row_hbm, col_hbm, st_hbm, tt_hbm, pk_hbm,
                    row_v, col_v, st_v, tt_v, local):
    c = lax.axis_index("c")
    s = lax.axis_index("s")
    wid = s * NC + c
    r0, trips = _chunk_range(wid, ROWS // NW, ROWS % NW)

    iota = lax.iota(jnp.int32, 16)
    neg1 = jnp.full((16,), -1, jnp.int32)

    def init_body(i, _):
        local[pl.ds(i * 16, 16)] = neg1
        return 0

    lax.fori_loop(0, N // 16, init_body, 0)

    perms = [(iota + k) % 16 for k in range(1, 16)]

    def upd(idx, val):
        for p in perms:
            oi = _gather16(idx, p)
            ov = _gather16(val, p)
            val = jnp.where(oi == idx, jnp.maximum(val, ov), val)
        cur = plsc.load_gather(local, [idx])
        plsc.store_scatter(local, [idx], jnp.maximum(cur, val))

    def sub_body(sub, _):
        r = r0 + sub
        e0 = r * SUB
        pltpu.sync_copy(row_hbm.at[pl.ds(e0, SUB)], row_v)
        pltpu.sync_copy(col_hbm.at[pl.ds(e0, SUB)], col_v)
        pltpu.sync_copy(st_hbm.at[pl.ds(e0, SUB)], st_v)
        pltpu.sync_copy(tt_hbm.at[pl.ds(e0, SUB)], tt_v)

        def batch_body(j, _):
            eid = e0 + j * 16 + iota
            ridx = row_v[pl.ds(j * 16, 16)]
            sval = st_v[pl.ds(j * 16, 16)]
            upd(ridx, eid * 8 + sval)
            cidx = col_v[pl.ds(j * 16, 16)]
            tval = tt_v[pl.ds(j * 16, 16)]
            upd(cidx, (eid + E) * 8 + tval)
            return 0

        lax.fori_loop(0, SUB // 16, batch_body, 0)
        return 0

    lax.fori_loop(0, trips, sub_body, 0)
    pltpu.sync_copy(local, pk_hbm.at[pl.ds(wid * N, N)])


def _node_type(row, col, st, tt):
    mesh = plsc.VectorSubcoreMesh(core_axis_name="c", subcore_axis_name="s")
    return pl.kernel(
        _node_type_body,
        out_type=jax.ShapeDtypeStruct((NW * N,), jnp.int32),
        mesh=mesh,
        compiler_params=pltpu.CompilerParams(use_tc_tiling_on_sc=False, needs_layout_passes=False),
        scratch_types=[
            pltpu.VMEM((SUB,), jnp.int32),
            pltpu.VMEM((SUB,), jnp.int32),
            pltpu.VMEM((SUB,), jnp.int32),
            pltpu.VMEM((SUB,), jnp.int32),
            pltpu.VMEM((N,), jnp.int32),
        ],
    )(row, col, st, tt)


_BF = 1000


def _final_body(u_ref, s_ref, packed_ref, x_ref, wa2_ref, ba_ref, lng_ref,
                lnb_ref, out_ref):
    S = s_ref[0] + s_ref[1]
    u0 = u_ref[0]
    u1 = u_ref[1]
    chunks = []
    for h in range(H):
        den_f = S[:, h:h + 1] + 1e-16
        den_b = S[:, h + 4:h + 5] + 1e-16
        chunks.append(u0[:, h * DK:(h + 1) * DK] / den_f
                      + u1[:, h * DK:(h + 1) * DK] / den_b)
    a = jnp.concatenate(chunks, axis=1)
    packed = jnp.max(packed_ref[0], axis=1, keepdims=True)
    nt = jnp.where(packed >= 0, packed & 7, 0)
    g = jax.nn.gelu(a)
    x = x_ref[...]
    acc = jnp.zeros_like(x)
    gam = jnp.zeros_like(x)
    bet = jnp.zeros_like(x)
    for t in range(T):
        yt = lax.dot_general(g, wa2_ref[t], (((1,), (1,)), ((), ()))) + ba_ref[t][None, :]
        sel = nt == t
        acc = jnp.where(sel, yt, acc)
        gam = jnp.where(sel, lng_ref[t][None, :], gam)
        bet = jnp.where(sel, lnb_ref[t][None, :], bet)
    h = acc + x
    mu = jnp.mean(h, axis=-1, keepdims=True)
    var = jnp.mean((h - mu) ** 2, axis=-1, keepdims=True)
    out_ref[...] = (h - mu) / jnp.sqrt(var + 1e-5) * gam + bet


def _final_stage(U, S_parts, packed_parts, x, Wa2, ba, ln_g, ln_b):
    nb = N // _BF
    P2 = packed_parts.shape[2]
    return pl.pallas_call(
        _final_body,
        grid=(nb,),
        in_specs=[
            pl.BlockSpec((2, _BF, IN), lambda b: (0, b, 0)),
            pl.BlockSpec((2, _BF, 8), lambda b: (0, b, 0)),
            pl.BlockSpec((1, _BF, P2), lambda b: (b, 0, 0)),
            pl.BlockSpec((_BF, IN), lambda b: (b, 0)),
            pl.BlockSpec((T, IN, OUT), lambda b: (0, 0, 0)),
            pl.BlockSpec((T, IN), lambda b: (0, 0)),
            pl.BlockSpec((T, IN), lambda b: (0, 0)),
            pl.BlockSpec((T, IN), lambda b: (0, 0)),
        ],
        out_specs=pl.BlockSpec((_BF, IN), lambda b: (b, 0)),
        out_shape=jax.ShapeDtypeStruct((N, IN), jnp.float32),
    )(U, S_parts, packed_parts, x, Wa2, ba, ln_g, ln_b)


def kernel(x, edge_index, edge_attr, Wk, bk, Wq, bq, Wv, bv, Wa, ba, ln_g, ln_b):
    row = edge_index[0].astype(jnp.int32)
    col = edge_index[1].astype(jnp.int32)
    src_t = edge_attr[:, 3].astype(jnp.int32)
    trg_t = edge_attr[:, 4].astype(jnp.int32)

    KQ, V = _make_tables(x, Wk, bk, Wq, bq, Wv, bv)

    i1 = src_t * N + row
    i2 = trg_t * N + col
    zeros = jnp.zeros((N, OUT), jnp.float32)
    zeros8 = jnp.zeros((N, 8), jnp.float32)

    idx4 = jnp.stack([i1, i2, col, row])
    fb, S_parts = _edge_att(KQ, idx4, zeros8)
    U = _msg_aggr(V, i1, i2, col, row, fb, zeros)
    pk = _node_type(row, col, src_t, trg_t)
    packed_parts = pk.reshape(NW, N // _BF, _BF).transpose(1, 2, 0)

    Wa2 = Wa[:, :, :OUT] + Wa[:, :, OUT:]
    return _final_stage(U, S_parts, packed_parts, x, Wa2, ba, ln_g, ln_b)


# speedup vs baseline: 16.4555x; 1.0093x over previous
import functools
import math

import jax
import jax.numpy as jnp
from jax import lax
from jax.experimental import pallas as pl
from jax.experimental.pallas import tpu as pltpu
from jax.experimental.pallas import tpu_sc as plsc

N = 10000
T = 8
H = 4
IN = 128
OUT = 128
DK = OUT // H
E = 320000

NC = 2
NS = 16
NW = NC * NS
SUB = 80
EW = E // NW
NSUB = EW // SUB
ROWS = E // SUB


_BN = 1000


def _tables_body(x_ref, wk_ref, bk_ref, wq_ref, bq_ref, wv_ref, bv_ref,
                 kq_ref, v_ref):
    x = x_ref[...]
    t = pl.program_id(0)
    inv = 1.0 / math.sqrt(DK)
    bk = bk_ref[pl.ds(t, 1), :]
    bq = bq_ref[pl.ds(t, 1), :]
    bv = bv_ref[pl.ds(t, 1), :]
    k = (lax.dot_general(x, wk_ref[0], (((1,), (1,)), ((), ()))) + bk) * inv
    q = lax.dot_general(x, wq_ref[0], (((1,), (1,)), ((), ()))) + bq
    v = lax.dot_general(x, wv_ref[0], (((1,), (1,)), ((), ()))) + bv
    kq_ref[...] = jnp.concatenate([k, q], axis=1)
    v_ref[...] = v


def _make_tables(x, Wk, bk, Wq, bq, Wv, bv):
    nb = N // _BN
    return pl.pallas_call(
        _tables_body,
        grid=(T, nb),
        in_specs=[
            pl.BlockSpec((_BN, IN), lambda t, b: (b, 0)),
            pl.BlockSpec((1, OUT, IN), lambda t, b: (t, 0, 0)),
            pl.BlockSpec((T, OUT), lambda t, b: (0, 0)),
            pl.BlockSpec((1, OUT, IN), lambda t, b: (t, 0, 0)),
            pl.BlockSpec((T, OUT), lambda t, b: (0, 0)),
            pl.BlockSpec((1, OUT, IN), lambda t, b: (t, 0, 0)),
            pl.BlockSpec((T, OUT), lambda t, b: (0, 0)),
        ],
        out_specs=[
            pl.BlockSpec((_BN, 2 * OUT), lambda t, b: (t * (N // _BN) + b, 0)),
            pl.BlockSpec((_BN, OUT), lambda t, b: (t * (N // _BN) + b, 0)),
        ],
        out_shape=[
            jax.ShapeDtypeStruct((T * N, 2 * OUT), jnp.float32),
            jax.ShapeDtypeStruct((T * N, OUT), jnp.float32),
        ],
    )(x, Wk, bk, Wq, bq, Wv, bv)


SUB = 128
ROWS = E // SUB


def _chunk_range(w, per, extra):
    r0 = w * per + jnp.minimum(w, extra)
    trips = per + jnp.where(w < extra, 1, 0)
    return r0, trips


def _edge_att_body(kq_hbm, idx4_hbm, zeros_hbm,
                   fb_hbm, s_hbm,
                   idx4_v, kq1_b, kq2_b,
                   f_stage, b_stage, sfs, sbs, s_sp, sem1, sem2,
                   sem_w, sem_a):
    c = lax.axis_index("c")
    s = lax.axis_index("s")
    wid = s * NC + c
    r0, trips = _chunk_range(wid, ROWS // NW, ROWS % NW)

    @pl.when(s == 0)
    def _():
        pltpu.sync_copy(zeros_hbm, s_sp)

    pltpu.sync_copy(zeros_hbm.at[pl.ds(0, SUB)], sfs.at[0])
    pltpu.sync_copy(zeros_hbm.at[pl.ds(0, SUB)], sfs.at[1])
    pltpu.sync_copy(zeros_hbm.at[pl.ds(0, SUB)], sbs.at[0])
    pltpu.sync_copy(zeros_hbm.at[pl.ds(0, SUB)], sbs.at[1])
    plsc.subcore_barrier()

    iota = lax.iota(jnp.int32, 16)

    def sub_body(sub, _):
        r = r0 + sub
        e0 = r * SUB
        p = sub % 2
        i4p = idx4_v.at[p]
        fsp = f_stage.at[p]
        bsp = b_stage.at[p]
        sfp = sfs.at[p]
        sbp = sbs.at[p]
        @pl.when(sub >= 2)
        def _():
            pltpu.make_async_copy(fsp, fb_hbm.at[0, :, pl.ds(e0, SUB)],
                                  sem_w).wait()
            pltpu.make_async_copy(bsp, fb_hbm.at[1, :, pl.ds(e0, SUB)],
                                  sem_w).wait()
            pltpu.make_async_copy(sfp, s_sp.at[idx4_v.at[p, 2]], sem_a).wait()
            pltpu.make_async_copy(sbp, s_sp.at[idx4_v.at[p, 3]], sem_a).wait()

        pltpu.sync_copy(idx4_hbm.at[:, pl.ds(e0, SUB)], i4p)
        cp1 = pltpu.async_copy(kq_hbm.at[idx4_v.at[p, 0]], kq1_b, sem1)
        cp2 = pltpu.async_copy(kq_hbm.at[idx4_v.at[p, 1]], kq2_b, sem2)
        cp1.wait()
        cp2.wait()

        def batch_body(j, _):
            eidx = iota + j * 16
            facc = [jnp.zeros((16,), jnp.float32) for _ in range(H)]
            bacc = [jnp.zeros((16,), jnp.float32) for _ in range(H)]
            for d in range(DK):
                rot = (iota + d) & (DK - 1)
                qg1 = [plsc.load_gather(kq1_b, [eidx, rot + (OUT + g * DK)])
                       for g in range(H)]
                qs1 = (qg1[0] + qg1[1]) + (qg1[2] + qg1[3])
                qg2 = [plsc.load_gather(kq2_b, [eidx, rot + (OUT + g * DK)])
                       for g in range(H)]
                qs2 = (qg2[0] + qg2[1]) + (qg2[2] + qg2[3])
                for h in range(H):
                    ch = rot + h * DK
                    k1 = plsc.load_gather(kq1_b, [eidx, ch])
                    facc[h] = facc[h] + k1 * qs2
                    k2 = plsc.load_gather(kq2_b, [eidx, ch])
                    bacc[h] = bacc[h] + k2 * qs1
            for h in range(H):
                fh = jnp.exp(facc[h])
                bh = jnp.exp(bacc[h])
                fsp[h, pl.ds(j * 16, 16)] = fh
                bsp[h, pl.ds(j * 16, 16)] = bh
                ch = jnp.full((16,), h, jnp.int32)
                ch4 = jnp.full((16,), h + 4, jnp.int32)
                plsc.store_scatter(sfp, [eidx, ch], fh)
                plsc.store_scatter(sbp, [eidx, ch4], bh)
            return 0

        lax.fori_loop(0, SUB // 16, batch_body, 0)
        pltpu.async_copy(fsp, fb_hbm.at[0, :, pl.ds(e0, SUB)], sem_w)
        pltpu.async_copy(bsp, fb_hbm.at[1, :, pl.ds(e0, SUB)], sem_w)
        pltpu.async_copy(sfp, s_sp.at[idx4_v.at[p, 2]], sem_a, add=True)
        pltpu.async_copy(sbp, s_sp.at[idx4_v.at[p, 3]], sem_a, add=True)
        return 0

    lax.fori_loop(0, trips, sub_body, 0)
    for q in (0, 1):
        pltpu.make_async_copy(f_stage.at[q], fb_hbm.at[0, :, pl.ds(0, SUB)],
                              sem_w).wait()
        pltpu.make_async_copy(b_stage.at[q], fb_hbm.at[1, :, pl.ds(0, SUB)],
                              sem_w).wait()
        pltpu.make_async_copy(sfs.at[q], s_sp.at[idx4_v.at[q, 2]],
                              sem_a).wait()
        pltpu.make_async_copy(sbs.at[q], s_sp.at[idx4_v.at[q, 3]],
                              sem_a).wait()
    plsc.subcore_barrier()

    @pl.when(s == 0)
    def _():
        pltpu.sync_copy(s_sp, s_hbm.at[c])


def _edge_att(KQ, idx4, zeros8):
    mesh = plsc.VectorSubcoreMesh(core_axis_name="c", subcore_axis_name="s")
    return pl.kernel(
        _edge_att_body,
        out_type=[
            jax.ShapeDtypeStruct((2, H, E), jnp.float32),
            jax.ShapeDtypeStruct((2, N, 8), jnp.float32),
        ],
        mesh=mesh,
        compiler_params=pltpu.CompilerParams(use_tc_tiling_on_sc=False, needs_layout_passes=False),
        scratch_types=[
            pltpu.VMEM((2, 4, SUB), jnp.int32),
            pltpu.VMEM((SUB, 2 * OUT), jnp.float32),
            pltpu.VMEM((SUB, 2 * OUT), jnp.float32),
            pltpu.VMEM((2, H, SUB), jnp.float32),
            pltpu.VMEM((2, H, SUB), jnp.float32),
            pltpu.VMEM((2, SUB, 8), jnp.float32),
            pltpu.VMEM((2, SUB, 8), jnp.float32),
            pltpu.VMEM_SHARED((N, 8), jnp.float32),
            pltpu.SemaphoreType.DMA,
            pltpu.SemaphoreType.DMA,
            pltpu.SemaphoreType.DMA,
            pltpu.SemaphoreType.DMA,
        ],
    )(KQ, idx4, zeros8)


SUB2 = 128
ROWS2 = E // SUB2


def _msg_body(v_hbm, i1_hbm, i2_hbm, col_hbm, row_hbm, fb_hbm, zeros_hbm,
              u_hbm, idx_v, seg, fb_v, v_b, u_sp,
              semg0, semg1, sema0, sema1):
    c = lax.axis_index("c")
    s = lax.axis_index("s")
    r0, trips = _chunk_range(s, ROWS2 // NS, ROWS2 % NS)

    @pl.when(s == 0)
    def _():
        pltpu.sync_copy(zeros_hbm, u_sp)

    plsc.subcore_barrier()
    iota = lax.iota(jnp.int32, 16)

    def direction(d_ix, ix_hbm, seg_hbm):
        def issue(sub, p):
            e0 = (r0 + sub) * SUB2
            pltpu.sync_copy(ix_hbm.at[pl.ds(e0, SUB2)], idx_v.at[p])
            if p == 0:
                pltpu.async_copy(v_hbm.at[idx_v.at[0]], v_b.at[0], semg0)
            else:
                pltpu.async_copy(v_hbm.at[idx_v.at[1]], v_b.at[1], semg1)

        issue(0, 0)

        def sub_body(sub, _):
            r = r0 + sub
            e0 = r * SUB2
            p = sub % 2

            @pl.when((sub + 1 < trips) & (p == 0))
            def _():
                @pl.when(sub >= 1)
                def _():
                    pltpu.make_async_copy(v_b.at[1], u_sp.at[seg.at[1]],
                                          sema1).wait()
                issue(sub + 1, 1)

            @pl.when((sub + 1 < trips) & (p == 1))
            def _():
                pltpu.make_async_copy(v_b.at[0], u_sp.at[seg.at[0]],
                                      sema0).wait()
                issue(sub + 1, 0)

            @pl.when(p == 0)
            def _():
                pltpu.make_async_copy(v_hbm.at[idx_v.at[0]], v_b.at[0],
                                      semg0).wait()

            @pl.when(p == 1)
            def _():
                pltpu.make_async_copy(v_hbm.at[idx_v.at[1]], v_b.at[1],
                                      semg1).wait()

            pltpu.sync_copy(fb_hbm.at[d_ix, :, pl.ds(e0, SUB2)], fb_v)
            vbp = v_b.at[p]

            def batch_body(j, _):
                eidx = iota + j * 16
                a = [fb_v[h, pl.ds(j * 16, 16)] for h in range(H)]
                for d in range(DK):
                    rot = (iota + d) & (DK - 1)
                    for h in range(H):
                        cd = rot + h * DK
                        vd = plsc.load_gather(vbp, [eidx, cd])
                        plsc.store_scatter(vbp, [eidx, cd], vd * a[h])
                return 0

            lax.fori_loop(0, SUB2 // 16, batch_body, 0)

            @pl.when(p == 0)
            def _():
                pltpu.sync_copy(seg_hbm.at[pl.ds(e0, SUB2)], seg.at[0])
                pltpu.async_copy(v_b.at[0], u_sp.at[seg.at[0]], sema0,
                                 add=True)

            @pl.when(p == 1)
            def _():
                pltpu.sync_copy(seg_hbm.at[pl.ds(e0, SUB2)], seg.at[1])
                pltpu.async_copy(v_b.at[1], u_sp.at[seg.at[1]], sema1,
                                 add=True)

            return 0

        lax.fori_loop(0, trips, sub_body, 0)
        pltpu.make_async_copy(v_b.at[0], u_sp.at[seg.at[0]], sema0).wait()
        pltpu.make_async_copy(v_b.at[1], u_sp.at[seg.at[1]], sema1).wait()

    @pl.when(c == 0)
    def _():
        direction(0, i1_hbm, col_hbm)

    @pl.when(c == 1)
    def _():
        direction(1, i2_hbm, row_hbm)

    plsc.subcore_barrier()

    @pl.when(s == 0)
    def _():
        pltpu.sync_copy(u_sp, u_hbm.at[c])


def _msg_aggr(V, i1, i2, col, row, fb, zeros):
    mesh = plsc.VectorSubcoreMesh(core_axis_name="c", subcore_axis_name="s")
    return pl.kernel(
        _msg_body,
        out_type=jax.ShapeDtypeStruct((2, N, OUT), jnp.float32),
        mesh=mesh,
        compiler_params=pltpu.CompilerParams(use_tc_tiling_on_sc=False, needs_layout_passes=False),
        scratch_types=[
            pltpu.VMEM((2, SUB2), jnp.int32),
            pltpu.VMEM((2, SUB2), jnp.int32),
            pltpu.VMEM((H, SUB2), jnp.float32),
            pltpu.VMEM((2, SUB2, OUT), jnp.float32),
            pltpu.VMEM_SHARED((N, OUT), jnp.float32),
            pltpu.SemaphoreType.DMA,
            pltpu.SemaphoreType.DMA,
            pltpu.SemaphoreType.DMA,
            pltpu.SemaphoreType.DMA,
        ],
    )(V, i1, i2, col, row, fb, zeros)


def _gather16(x, idx):
    dn = lax.GatherDimensionNumbers(
        offset_dims=(), collapsed_slice_dims=(0,), start_index_map=(0,))
    return lax.gather(x, idx[:, None], dn, (1,),
                      mode=lax.GatherScatterMode.PROMISE_IN_BOUNDS)


def _node_type_body(row_hbm, col_hbm, st_hbm, tt_hbm, pk_hbm,
                    row_v, col_v, st_v, tt_v, local):
    c = lax.axis_index("c")
    s = lax.axis_index("s")
    wid = s * NC + c
    r0, trips = _chunk_range(wid, ROWS // NW, ROWS % NW)

    iota = lax.iota(jnp.int32, 16)
    neg1 = jnp.full((16,), -1, jnp.int32)

    def init_body(i, _):
        local[pl.ds(i * 16, 16)] = neg1
        return 0

    lax.fori_loop(0, N // 16, init_body, 0)

    perms = [(iota + k) % 16 for k in range(1, 16)]

    def upd(idx, val):
        for p in perms:
            oi = _gather16(idx, p)
            ov = _gather16(val, p)
            val = jnp.where(oi == idx, jnp.maximum(val, ov), val)
        cur = plsc.load_gather(local, [idx])
        plsc.store_scatter(local, [idx], jnp.maximum(cur, val))

    def sub_body(sub, _):
        r = r0 + sub
        e0 = r * SUB
        pltpu.sync_copy(row_hbm.at[pl.ds(e0, SUB)], row_v)
        pltpu.sync_copy(col_hbm.at[pl.ds(e0, SUB)], col_v)
        pltpu.sync_copy(st_hbm.at[pl.ds(e0, SUB)], st_v)
        pltpu.sync_copy(tt_hbm.at[pl.ds(e0, SUB)], tt_v)

        def batch_body(j, _):
            eid = e0 + j * 16 + iota
            ridx = row_v[pl.ds(j * 16, 16)]
            sval = st_v[pl.ds(j * 16, 16)]
            upd(ridx, eid * 8 + sval)
            cidx = col_v[pl.ds(j * 16, 16)]
            tval = tt_v[pl.ds(j * 16, 16)]
            upd(cidx, (eid + E) * 8 + tval)
            return 0

        lax.fori_loop(0, SUB // 16, batch_body, 0)
        return 0

    lax.fori_loop(0, trips, sub_body, 0)
    pltpu.sync_copy(local, pk_hbm.at[pl.ds(wid * N, N)])


def _node_type(row, col, st, tt):
    mesh = plsc.VectorSubcoreMesh(core_axis_name="c", subcore_axis_name="s")
    return pl.kernel(
        _node_type_body,
        out_type=jax.ShapeDtypeStruct((NW * N,), jnp.int32),
        mesh=mesh,
        compiler_params=pltpu.CompilerParams(use_tc_tiling_on_sc=False, needs_layout_passes=False),
        scratch_types=[
            pltpu.VMEM((SUB,), jnp.int32),
            pltpu.VMEM((SUB,), jnp.int32),
            pltpu.VMEM((SUB,), jnp.int32),
            pltpu.VMEM((SUB,), jnp.int32),
            pltpu.VMEM((N,), jnp.int32),
        ],
    )(row, col, st, tt)


_BF = 1000


def _final_body(u_ref, s_ref, packed_ref, x_ref, wa2_ref, ba_ref, lng_ref,
                lnb_ref, out_ref):
    S = s_ref[0] + s_ref[1]
    u0 = u_ref[0]
    u1 = u_ref[1]
    chunks = []
    for h in range(H):
        den_f = S[:, h:h + 1] + 1e-16
        den_b = S[:, h + 4:h + 5] + 1e-16
        chunks.append(u0[:, h * DK:(h + 1) * DK] / den_f
                      + u1[:, h * DK:(h + 1) * DK] / den_b)
    a = jnp.concatenate(chunks, axis=1)
    packed = jnp.max(packed_ref[0], axis=1, keepdims=True)
    nt = jnp.where(packed >= 0, packed & 7, 0)
    g = jax.nn.gelu(a)
    x = x_ref[...]
    acc = jnp.zeros_like(x)
    gam = jnp.zeros_like(x)
    bet = jnp.zeros_like(x)
    for t in range(T):
        yt = lax.dot_general(g, wa2_ref[t], (((1,), (1,)), ((), ()))) + ba_ref[t][None, :]
        sel = nt == t
        acc = jnp.where(sel, yt, acc)
        gam = jnp.where(sel, lng_ref[t][None, :], gam)
        bet = jnp.where(sel, lnb_ref[t][None, :], bet)
    h = acc + x
    mu = jnp.mean(h, axis=-1, keepdims=True)
    var = jnp.mean((h - mu) ** 2, axis=-1, keepdims=True)
    out_ref[...] = (h - mu) / jnp.sqrt(var + 1e-5) * gam + bet


def _final_stage(U, S_parts, packed_parts, x, Wa2, ba, ln_g, ln_b):
    nb = N // _BF
    P2 = packed_parts.shape[2]
    return pl.pallas_call(
        _final_body,
        grid=(nb,),
        in_specs=[
            pl.BlockSpec((2, _BF, IN), lambda b: (0, b, 0)),
            pl.BlockSpec((2, _BF, 8), lambda b: (0, b, 0)),
            pl.BlockSpec((1, _BF, P2), lambda b: (b, 0, 0)),
            pl.BlockSpec((_BF, IN), lambda b: (b, 0)),
            pl.BlockSpec((T, IN, OUT), lambda b: (0, 0, 0)),
            pl.BlockSpec((T, IN), lambda b: (0, 0)),
            pl.BlockSpec((T, IN), lambda b: (0, 0)),
            pl.BlockSpec((T, IN), lambda b: (0, 0)),
        ],
        out_specs=pl.BlockSpec((_BF, IN), lambda b: (b, 0)),
        out_shape=jax.ShapeDtypeStruct((N, IN), jnp.float32),
    )(U, S_parts, packed_parts, x, Wa2, ba, ln_g, ln_b)


def kernel(x, edge_index, edge_attr, Wk, bk, Wq, bq, Wv, bv, Wa, ba, ln_g, ln_b):
    row = edge_index[0].astype(jnp.int32)
    col = edge_index[1].astype(jnp.int32)
    src_t = edge_attr[:, 3].astype(jnp.int32)
    trg_t = edge_attr[:, 4].astype(jnp.int32)

    KQ, V = _make_tables(x, Wk, bk, Wq, bq, Wv, bv)

    i1 = src_t * N + row
    i2 = trg_t * N + col
    zeros = jnp.zeros((N, OUT), jnp.float32)
    zeros8 = jnp.zeros((N, 8), jnp.float32)

    idx4 = jnp.stack([i1, i2, col, row])
    fb, S_parts = _edge_att(KQ, idx4, zeros8)
    U = _msg_aggr(V, i1, i2, col, row, fb, zeros)
    pk = _node_type(row, col, src_t, trg_t)
    packed_parts = pk.reshape(NW, N // _BF, _BF).transpose(1, 2, 0)

    Wa2 = Wa[:, :, :OUT] + Wa[:, :, OUT:]
    return _final_stage(U, S_parts, packed_parts, x, Wa2, ba, ln_g, ln_b)


# speedup vs baseline: 16.4600x; 1.0003x over previous
import math

import jax
import jax.numpy as jnp
from jax import lax
from jax.experimental import pallas as pl
from jax.experimental.pallas import tpu as pltpu
from jax.experimental.pallas import tpu_sc as plsc

N = 10000
T = 8
H = 4
IN = 128
OUT = 128
DK = OUT // H
E = 320000

NC = 2
NS = 16
NW = NC * NS
SUB = 80
EW = E // NW
NSUB = EW // SUB
ROWS = E // SUB


_BN = 1000


def _tables_body(x_ref, wk_ref, bk_ref, wq_ref, bq_ref, wv_ref, bv_ref,
                 kq_ref, v_ref):
    x = x_ref[...]
    t = pl.program_id(0)
    inv = 1.0 / math.sqrt(DK)
    bk = bk_ref[pl.ds(t, 1), :]
    bq = bq_ref[pl.ds(t, 1), :]
    bv = bv_ref[pl.ds(t, 1), :]
    k = (lax.dot_general(x, wk_ref[0], (((1,), (1,)), ((), ()))) + bk) * inv
    q = lax.dot_general(x, wq_ref[0], (((1,), (1,)), ((), ()))) + bq
    v = lax.dot_general(x, wv_ref[0], (((1,), (1,)), ((), ()))) + bv
    kq_ref[...] = jnp.concatenate([k, q], axis=1)
    v_ref[...] = v


def _make_tables(x, Wk, bk, Wq, bq, Wv, bv):
    nb = N // _BN
    return pl.pallas_call(
        _tables_body,
        grid=(T, nb),
        in_specs=[
            pl.BlockSpec((_BN, IN), lambda t, b: (b, 0)),
            pl.BlockSpec((1, OUT, IN), lambda t, b: (t, 0, 0)),
            pl.BlockSpec((T, OUT), lambda t, b: (0, 0)),
            pl.BlockSpec((1, OUT, IN), lambda t, b: (t, 0, 0)),
            pl.BlockSpec((T, OUT), lambda t, b: (0, 0)),
            pl.BlockSpec((1, OUT, IN), lambda t, b: (t, 0, 0)),
            pl.BlockSpec((T, OUT), lambda t, b: (0, 0)),
        ],
        out_specs=[
            pl.BlockSpec((_BN, 2 * OUT), lambda t, b: (t * (N // _BN) + b, 0)),
            pl.BlockSpec((_BN, OUT), lambda t, b: (t * (N // _BN) + b, 0)),
        ],
        out_shape=[
            jax.ShapeDtypeStruct((T * N, 2 * OUT), jnp.float32),
            jax.ShapeDtypeStruct((T * N, OUT), jnp.float32),
        ],
    )(x, Wk, bk, Wq, bq, Wv, bv)


SUB = 128
ROWS = E // SUB


def _chunk_range(w, per, extra):
    r0 = w * per + jnp.minimum(w, extra)
    trips = per + jnp.where(w < extra, 1, 0)
    return r0, trips


def _edge_att_body(kq_hbm, idx4_hbm, zeros_hbm,
                   fb_hbm, s_hbm,
                   idx4_v, kq1_b, kq2_b,
                   f_stage, b_stage, sfs, sbs, s_sp, sem1, sem2,
                   sem_w, sem_a):
    c = lax.axis_index("c")
    s = lax.axis_index("s")
    wid = s * NC + c
    r0, trips = _chunk_range(wid, ROWS // NW, ROWS % NW)

    @pl.when(s == 0)
    def _():
        pltpu.sync_copy(zeros_hbm, s_sp)

    pltpu.sync_copy(zeros_hbm.at[pl.ds(0, SUB)], sfs.at[0])
    pltpu.sync_copy(zeros_hbm.at[pl.ds(0, SUB)], sfs.at[1])
    pltpu.sync_copy(zeros_hbm.at[pl.ds(0, SUB)], sbs.at[0])
    pltpu.sync_copy(zeros_hbm.at[pl.ds(0, SUB)], sbs.at[1])
    plsc.subcore_barrier()

    iota = lax.iota(jnp.int32, 16)

    def sub_body(sub, _):
        r = r0 + sub
        e0 = r * SUB
        p = sub % 2
        i4p = idx4_v.at[p]
        fsp = f_stage.at[p]
        bsp = b_stage.at[p]
        sfp = sfs.at[p]
        sbp = sbs.at[p]
        @pl.when(sub >= 2)
        def _():
            pltpu.make_async_copy(fsp, fb_hbm.at[0, :, pl.ds(e0, SUB)],
                                  sem_w).wait()
            pltpu.make_async_copy(bsp, fb_hbm.at[1, :, pl.ds(e0, SUB)],
                                  sem_w).wait()
            pltpu.make_async_copy(sfp, s_sp.at[idx4_v.at[p, 2]], sem_a).wait()
            pltpu.make_async_copy(sbp, s_sp.at[idx4_v.at[p, 3]], sem_a).wait()

        pltpu.sync_copy(idx4_hbm.at[:, pl.ds(e0, SUB)], i4p)
        cp1 = pltpu.async_copy(kq_hbm.at[idx4_v.at[p, 0]], kq1_b, sem1)
        cp2 = pltpu.async_copy(kq_hbm.at[idx4_v.at[p, 1]], kq2_b, sem2)
        cp1.wait()
        cp2.wait()

        def batch_body(j, _):
            eidx = iota + j * 16
            facc = [jnp.zeros((16,), jnp.float32) for _ in range(H)]
            bacc = [jnp.zeros((16,), jnp.float32) for _ in range(H)]
            for d in range(DK):
                rot = (iota + d) & (DK - 1)
                qg1 = [plsc.load_gather(kq1_b, [eidx, rot + (OUT + g * DK)])
                       for g in range(H)]
                qs1 = (qg1[0] + qg1[1]) + (qg1[2] + qg1[3])
                qg2 = [plsc.load_gather(kq2_b, [eidx, rot + (OUT + g * DK)])
                       for g in range(H)]
                qs2 = (qg2[0] + qg2[1]) + (qg2[2] + qg2[3])
                for h in range(H):
                    ch = rot + h * DK
                    k1 = plsc.load_gather(kq1_b, [eidx, ch])
                    facc[h] = facc[h] + k1 * qs2
                    k2 = plsc.load_gather(kq2_b, [eidx, ch])
                    bacc[h] = bacc[h] + k2 * qs1
            for h in range(H):
                fh = jnp.exp(facc[h])
                bh = jnp.exp(bacc[h])
                fsp[h, pl.ds(j * 16, 16)] = fh
                bsp[h, pl.ds(j * 16, 16)] = bh
                ch = jnp.full((16,), h, jnp.int32)
                ch4 = jnp.full((16,), h + 4, jnp.int32)
                plsc.store_scatter(sfp, [eidx, ch], fh)
                plsc.store_scatter(sbp, [eidx, ch4], bh)
            return 0

        lax.fori_loop(0, SUB // 16, batch_body, 0)
        pltpu.async_copy(fsp, fb_hbm.at[0, :, pl.ds(e0, SUB)], sem_w)
        pltpu.async_copy(bsp, fb_hbm.at[1, :, pl.ds(e0, SUB)], sem_w)
        pltpu.async_copy(sfp, s_sp.at[idx4_v.at[p, 2]], sem_a, add=True)
        pltpu.async_copy(sbp, s_sp.at[idx4_v.at[p, 3]], sem_a, add=True)
        return 0

    lax.fori_loop(0, trips, sub_body, 0)
    for q in (0, 1):
        pltpu.make_async_copy(f_stage.at[q], fb_hbm.at[0, :, pl.ds(0, SUB)],
                              sem_w).wait()
        pltpu.make_async_copy(b_stage.at[q], fb_hbm.at[1, :, pl.ds(0, SUB)],
                              sem_w).wait()
        pltpu.make_async_copy(sfs.at[q], s_sp.at[idx4_v.at[q, 2]],
                              sem_a).wait()
        pltpu.make_async_copy(sbs.at[q], s_sp.at[idx4_v.at[q, 3]],
                              sem_a).wait()
    plsc.subcore_barrier()

    @pl.when(s == 0)
    def _():
        pltpu.sync_copy(s_sp, s_hbm.at[c])


def _edge_att(KQ, idx4, zeros8):
    mesh = plsc.VectorSubcoreMesh(core_axis_name="c", subcore_axis_name="s")
    return pl.kernel(
        _edge_att_body,
        out_type=[
            jax.ShapeDtypeStruct((2, H, E), jnp.float32),
            jax.ShapeDtypeStruct((2, N, 8), jnp.float32),
        ],
        mesh=mesh,
        compiler_params=pltpu.CompilerParams(use_tc_tiling_on_sc=False, needs_layout_passes=False),
        scratch_types=[
            pltpu.VMEM((2, 4, SUB), jnp.int32),
            pltpu.VMEM((SUB, 2 * OUT), jnp.float32),
            pltpu.VMEM((SUB, 2 * OUT), jnp.float32),
            pltpu.VMEM((2, H, SUB), jnp.float32),
            pltpu.VMEM((2, H, SUB), jnp.float32),
            pltpu.VMEM((2, SUB, 8), jnp.float32),
            pltpu.VMEM((2, SUB, 8), jnp.float32),
            pltpu.VMEM_SHARED((N, 8), jnp.float32),
            pltpu.SemaphoreType.DMA,
            pltpu.SemaphoreType.DMA,
            pltpu.SemaphoreType.DMA,
            pltpu.SemaphoreType.DMA,
        ],
    )(KQ, idx4, zeros8)


SUB2 = 128
ROWS2 = E // SUB2


def _msg_body(v_hbm, i1_hbm, i2_hbm, col_hbm, row_hbm, fb_hbm, zeros_hbm,
              u_hbm, idx_v, seg, fb_v, v_b, u_sp,
              semg0, semg1, sema0, sema1):
    c = lax.axis_index("c")
    s = lax.axis_index("s")
    r0, trips = _chunk_range(s, ROWS2 // NS, ROWS2 % NS)

    @pl.when(s == 0)
    def _():
        pltpu.sync_copy(zeros_hbm, u_sp)

    plsc.subcore_barrier()
    iota = lax.iota(jnp.int32, 16)

    def direction(d_ix, ix_hbm, seg_hbm):
        def issue(sub, p):
            e0 = (r0 + sub) * SUB2
            pltpu.sync_copy(ix_hbm.at[pl.ds(e0, SUB2)], idx_v.at[p])
            if p == 0:
                pltpu.async_copy(v_hbm.at[idx_v.at[0]], v_b.at[0], semg0)
            else:
                pltpu.async_copy(v_hbm.at[idx_v.at[1]], v_b.at[1], semg1)

        issue(0, 0)

        def sub_body(sub, _):
            r = r0 + sub
            e0 = r * SUB2
            p = sub % 2

            @pl.when((sub + 1 < trips) & (p == 0))
            def _():
                @pl.when(sub >= 1)
                def _():
                    pltpu.make_async_copy(v_b.at[1], u_sp.at[seg.at[1]],
                                          sema1).wait()
                issue(sub + 1, 1)

            @pl.when((sub + 1 < trips) & (p == 1))
            def _():
                pltpu.make_async_copy(v_b.at[0], u_sp.at[seg.at[0]],
                                      sema0).wait()
                issue(sub + 1, 0)

            @pl.when(p == 0)
            def _():
                pltpu.make_async_copy(v_hbm.at[idx_v.at[0]], v_b.at[0],
                                      semg0).wait()

            @pl.when(p == 1)
            def _():
                pltpu.make_async_copy(v_hbm.at[idx_v.at[1]], v_b.at[1],
                                      semg1).wait()

            pltpu.sync_copy(fb_hbm.at[d_ix, :, pl.ds(e0, SUB2)], fb_v)
            vbp = v_b.at[p]

            def batch_body(j, _):
                eidx = iota + j * 16
                a = [fb_v[h, pl.ds(j * 16, 16)] for h in range(H)]
                for d in range(DK):
                    rot = (iota + d) & (DK - 1)
                    for h in range(H):
                        cd = rot + h * DK
                        vd = plsc.load_gather(vbp, [eidx, cd])
                        plsc.store_scatter(vbp, [eidx, cd], vd * a[h])
                return 0

            lax.fori_loop(0, SUB2 // 16, batch_body, 0)

            @pl.when(p == 0)
            def _():
                pltpu.sync_copy(seg_hbm.at[pl.ds(e0, SUB2)], seg.at[0])
                pltpu.async_copy(v_b.at[0], u_sp.at[seg.at[0]], sema0,
                                 add=True)

            @pl.when(p == 1)
            def _():
                pltpu.sync_copy(seg_hbm.at[pl.ds(e0, SUB2)], seg.at[1])
                pltpu.async_copy(v_b.at[1], u_sp.at[seg.at[1]], sema1,
                                 add=True)

            return 0

        lax.fori_loop(0, trips, sub_body, 0)
        pltpu.make_async_copy(v_b.at[0], u_sp.at[seg.at[0]], sema0).wait()
        pltpu.make_async_copy(v_b.at[1], u_sp.at[seg.at[1]], sema1).wait()

    @pl.when(c == 0)
    def _():
        direction(0, i1_hbm, col_hbm)

    @pl.when(c == 1)
    def _():
        direction(1, i2_hbm, row_hbm)

    plsc.subcore_barrier()

    @pl.when(s == 0)
    def _():
        pltpu.sync_copy(u_sp, u_hbm.at[c])


def _msg_aggr(V, i1, i2, col, row, fb, zeros):
    mesh = plsc.VectorSubcoreMesh(core_axis_name="c", subcore_axis_name="s")
    return pl.kernel(
        _msg_body,
        out_type=jax.ShapeDtypeStruct((2, N, OUT), jnp.float32),
        mesh=mesh,
        compiler_params=pltpu.CompilerParams(use_tc_tiling_on_sc=False, needs_layout_passes=False),
        scratch_types=[
            pltpu.VMEM((2, SUB2), jnp.int32),
            pltpu.VMEM((2, SUB2), jnp.int32),
            pltpu.VMEM((H, SUB2), jnp.float32),
            pltpu.VMEM((2, SUB2, OUT), jnp.float32),
            pltpu.VMEM_SHARED((N, OUT), jnp.float32),
            pltpu.SemaphoreType.DMA,
            pltpu.SemaphoreType.DMA,
            pltpu.SemaphoreType.DMA,
            pltpu.SemaphoreType.DMA,
        ],
    )(V, i1, i2, col, row, fb, zeros)


def _gather16(x, idx):
    dn = lax.GatherDimensionNumbers(
        offset_dims=(), collapsed_slice_dims=(0,), start_index_map=(0,))
    return lax.gather(x, idx[:, None], dn, (1,),
                      mode=lax.GatherScatterMode.PROMISE_IN_BOUNDS)


def _node_type_body(row_hbm, col_hbm, st_hbm, tt_hbm, pk_hbm,
                    row_v, col_v, st_v, tt_v, local):
    c = lax.axis_index("c")
    s = lax.axis_index("s")
    wid = s * NC + c
    r0, trips = _chunk_range(wid, ROWS // NW, ROWS % NW)

    iota = lax.iota(jnp.int32, 16)
    neg1 = jnp.full((16,), -1, jnp.int32)

    def init_body(i, _):
        local[pl.ds(i * 16, 16)] = neg1
        return 0

    lax.fori_loop(0, N // 16, init_body, 0)

    perms = [(iota + k) % 16 for k in range(1, 16)]

    def upd(idx, val):
        for p in perms:
            oi = _gather16(idx, p)
            ov = _gather16(val, p)
            val = jnp.where(oi == idx, jnp.maximum(val, ov), val)
        cur = plsc.load_gather(local, [idx])
        plsc.store_scatter(local, [idx], jnp.maximum(cur, val))

    def sub_body(sub, _):
        r = r0 + sub
        e0 = r * SUB
        pltpu.sync_copy(row_hbm.at[pl.ds(e0, SUB)], row_v)
        pltpu.sync_copy(col_hbm.at[pl.ds(e0, SUB)], col_v)
        pltpu.sync_copy(st_hbm.at[pl.ds(e0, SUB)], st_v)
        pltpu.sync_copy(tt_hbm.at[pl.ds(e0, SUB)], tt_v)

        def batch_body(j, _):
            eid = e0 + j * 16 + iota
            ridx = row_v[pl.ds(j * 16, 16)]
            sval = st_v[pl.ds(j * 16, 16)]
            upd(ridx, eid * 8 + sval)
            cidx = col_v[pl.ds(j * 16, 16)]
            tval = tt_v[pl.ds(j * 16, 16)]
            upd(cidx, (eid + E) * 8 + tval)
            return 0

        lax.fori_loop(0, SUB // 16, batch_body, 0)
        return 0

    lax.fori_loop(0, trips, sub_body, 0)
    pltpu.sync_copy(local, pk_hbm.at[pl.ds(wid * N, N)])


def _node_type(row, col, st, tt):
    mesh = plsc.VectorSubcoreMesh(core_axis_name="c", subcore_axis_name="s")
    return pl.kernel(
        _node_type_body,
        out_type=jax.ShapeDtypeStruct((NW * N,), jnp.int32),
        mesh=mesh,
        compiler_params=pltpu.CompilerParams(use_tc_tiling_on_sc=False, needs_layout_passes=False),
        scratch_types=[
            pltpu.VMEM((SUB,), jnp.int32),
            pltpu.VMEM((SUB,), jnp.int32),
            pltpu.VMEM((SUB,), jnp.int32),
            pltpu.VMEM((SUB,), jnp.int32),
            pltpu.VMEM((N,), jnp.int32),
        ],
    )(row, col, st, tt)


_BF = 1000


def _final_body(u_ref, s_ref, packed_ref, x_ref, wa2_ref, ba_ref, lng_ref,
                lnb_ref, out_ref):
    S = s_ref[0] + s_ref[1]
    u0 = u_ref[0]
    u1 = u_ref[1]
    chunks = []
    for h in range(H):
        den_f = S[:, h:h + 1] + 1e-16
        den_b = S[:, h + 4:h + 5] + 1e-16
        chunks.append(u0[:, h * DK:(h + 1) * DK] / den_f
                      + u1[:, h * DK:(h + 1) * DK] / den_b)
    a = jnp.concatenate(chunks, axis=1)
    packed = jnp.max(packed_ref[0], axis=1, keepdims=True)
    nt = jnp.where(packed >= 0, packed & 7, 0)
    g = jax.nn.gelu(a)
    x = x_ref[...]
    acc = jnp.zeros_like(x)
    gam = jnp.zeros_like(x)
    bet = jnp.zeros_like(x)
    for t in range(T):
        yt = lax.dot_general(g, wa2_ref[t], (((1,), (1,)), ((), ()))) + ba_ref[t][None, :]
        sel = nt == t
        acc = jnp.where(sel, yt, acc)
        gam = jnp.where(sel, lng_ref[t][None, :], gam)
        bet = jnp.where(sel, lnb_ref[t][None, :], bet)
    h = acc + x
    mu = jnp.mean(h, axis=-1, keepdims=True)
    var = jnp.mean((h - mu) ** 2, axis=-1, keepdims=True)
    out_ref[...] = (h - mu) / jnp.sqrt(var + 1e-5) * gam + bet


def _final_stage(U, S_parts, packed_parts, x, Wa2, ba, ln_g, ln_b):
    nb = N // _BF
    P2 = packed_parts.shape[2]
    return pl.pallas_call(
        _final_body,
        grid=(nb,),
        in_specs=[
            pl.BlockSpec((2, _BF, IN), lambda b: (0, b, 0)),
            pl.BlockSpec((2, _BF, 8), lambda b: (0, b, 0)),
            pl.BlockSpec((1, _BF, P2), lambda b: (b, 0, 0)),
            pl.BlockSpec((_BF, IN), lambda b: (b, 0)),
            pl.BlockSpec((T, IN, OUT), lambda b: (0, 0, 0)),
            pl.BlockSpec((T, IN), lambda b: (0, 0)),
            pl.BlockSpec((T, IN), lambda b: (0, 0)),
            pl.BlockSpec((T, IN), lambda b: (0, 0)),
        ],
        out_specs=pl.BlockSpec((_BF, IN), lambda b: (b, 0)),
        out_shape=jax.ShapeDtypeStruct((N, IN), jnp.float32),
    )(U, S_parts, packed_parts, x, Wa2, ba, ln_g, ln_b)


def kernel(x, edge_index, edge_attr, Wk, bk, Wq, bq, Wv, bv, Wa, ba, ln_g, ln_b):
    row = edge_index[0].astype(jnp.int32)
    col = edge_index[1].astype(jnp.int32)
    src_t = edge_attr[:, 3].astype(jnp.int32)
    trg_t = edge_attr[:, 4].astype(jnp.int32)

    KQ, V = _make_tables(x, Wk, bk, Wq, bq, Wv, bv)

    i1 = src_t * N + row
    i2 = trg_t * N + col
    zeros = jnp.zeros((N, OUT), jnp.float32)
    zeros8 = jnp.zeros((N, 8), jnp.float32)

    idx4 = jnp.stack([i1, i2, col, row])
    fb, S_parts = _edge_att(KQ, idx4, zeros8)
    U = _msg_aggr(V, i1, i2, col, row, fb, zeros)
    pk = _node_type(row, col, src_t, trg_t)
    packed_parts = pk.reshape(NW, N // _BF, _BF).transpose(1, 2, 0)

    Wa2 = Wa[:, :, :OUT] + Wa[:, :, OUT:]
    return _final_stage(U, S_parts, packed_parts, x, Wa2, ba, ln_g, ln_b)
